# Initial kernel scaffold; baseline (speedup 1.0000x reference)
import jax
import jax.numpy as jnp
from jax.experimental import pallas as pl


def kernel(x, edge_index, edge_weight, W1, b1, W2, attn_l, attn_r, b2):
    raise NotImplementedError("write your pallas kernel here")


# speedup vs baseline: 9.9102x; 9.9102x over previous
import functools

import jax
import jax.numpy as jnp
from jax import lax
from jax.experimental import pallas as pl
from jax.experimental.pallas import tpu as pltpu
from jax.experimental.pallas import tpu_sc as plsc

N = 10000
E = 320000
D = 128
H = 8
O = 128

NC, NS = 2, 16
W = 80
ET2 = E // (NC * NS)
NW2 = ET2 // W
ET6 = E // NS
NW6 = ET6 // W
RCH = 624
HPC = H // NC
DQ = (HPC * N) // 8

_mesh = plsc.VectorSubcoreMesh(core_axis_name="c", subcore_axis_name="s")


def _row_chunks(sid, fn):
    off = pl.multiple_of(sid * RCH, 8)
    fn(off, RCH)

    @pl.when(sid == NS - 1)
    def _():
        fn(NS * RCH, N - NS * RCH)


@functools.partial(
    pl.kernel,
    out_type=(jax.ShapeDtypeStruct((N, D), jnp.float32),
              jax.ShapeDtypeStruct((N, D), jnp.float32)),
    mesh=_mesh,
    scratch_types=[
        pltpu.VMEM_SHARED((N, D), jnp.float32),
        pltpu.VMEM((W,), jnp.int32),
        pltpu.VMEM((W,), jnp.int32),
        pltpu.VMEM((W,), jnp.float32),
        pltpu.VMEM((W, D), jnp.float32),
    ],
)
def _agg0_kernel(h_hbm, src_hbm, dst_hbm, ew_hbm, zeros_hbm,
                 outa_hbm, outb_hbm, acc_sp, srcv, dstv, eww, rows):
    cid = lax.axis_index("c")
    sid = lax.axis_index("s")
    wid = cid * NS + sid
    _row_chunks(sid, lambda off, sz: pltpu.sync_copy(
        zeros_hbm.at[pl.ds(off, sz)], acc_sp.at[pl.ds(off, sz)]))
    plsc.subcore_barrier()

    def window(w, carry):
        base = pl.multiple_of(wid * ET2 + w * W, 8)
        pltpu.sync_copy(src_hbm.at[pl.ds(base, W)], srcv)
        pltpu.sync_copy(dst_hbm.at[pl.ds(base, W)], dstv)
        pltpu.sync_copy(ew_hbm.at[pl.ds(base, W)], eww)
        pltpu.sync_copy(h_hbm.at[srcv], rows)

        def scale(k, c2):
            w16 = eww[pl.ds(k * 16, 16)]
            for i in range(16):
                wgt = w16[i]
                row = k * 16 + i
                for j in range(D // 16):
                    sl = pl.ds(j * 16, 16)
                    rows[row, sl] = rows[row, sl] * wgt
            return c2
        lax.fori_loop(0, W // 16, scale, 0)
        pltpu.sync_copy(rows, acc_sp.at[dstv], add=True)
        return carry
    lax.fori_loop(0, NW2, window, 0)
    plsc.subcore_barrier()

    @pl.when(cid == 0)
    def _():
        _row_chunks(sid, lambda off, sz: pltpu.sync_copy(
            acc_sp.at[pl.ds(off, sz)], outa_hbm.at[pl.ds(off, sz)]))

    @pl.when(cid == 1)
    def _():
        _row_chunks(sid, lambda off, sz: pltpu.sync_copy(
            acc_sp.at[pl.ds(off, sz)], outb_hbm.at[pl.ds(off, sz)]))


@functools.partial(
    pl.kernel,
    out_type=(jax.ShapeDtypeStruct((HPC * N,), jnp.float32),
              jax.ShapeDtypeStruct((HPC * N,), jnp.float32)),
    mesh=_mesh,
    scratch_types=[
        pltpu.VMEM_SHARED((HPC * N,), jnp.float32),
        pltpu.VMEM_SHARED((HPC * N,), jnp.float32),
        pltpu.VMEM_SHARED((HPC * N,), jnp.float32),
        pltpu.VMEM((W,), jnp.int32),
        pltpu.VMEM((W,), jnp.int32),
        pltpu.VMEM((HPC, W), jnp.int32),
        pltpu.VMEM((HPC, W), jnp.int32),
        pltpu.VMEM((HPC, W), jnp.float32),
        pltpu.VMEM((HPC, W), jnp.float32),
        pltpu.VMEM((HPC, W), jnp.float32),
        pltpu.VMEM((HPC * 16,), jnp.float32),
        pltpu.VMEM((DQ,), jnp.float32),
    ],
)
def _den_kernel(elt_hbm, ert_hbm, m_hbm, src_hbm, dst_hbm, zeros_hbm,
                dena_hbm, denb_hbm,
                den_sp, el4_sp, er4_sp, srcv, dstv, idxs, idxd,
                elb, erb, eeb, mv, bnc):
    cid = lax.axis_index("c")
    sid = lax.axis_index("s")
    toff = pl.multiple_of(cid * (HPC * N), 8)
    moff = pl.multiple_of(cid * (HPC * 16), 8)
    pltpu.sync_copy(m_hbm.at[pl.ds(moff, HPC * 16)], mv)

    @pl.when(sid < 8)
    def _():
        qsl = pl.ds(pl.multiple_of(sid * DQ, 8), DQ)
        tsl = pl.ds(pl.multiple_of(toff + sid * DQ, 8), DQ)
        pltpu.sync_copy(zeros_hbm.at[qsl], bnc)
        pltpu.sync_copy(bnc, den_sp.at[qsl])
        pltpu.sync_copy(elt_hbm.at[tsl], bnc)
        pltpu.sync_copy(bnc, el4_sp.at[qsl])
        pltpu.sync_copy(ert_hbm.at[tsl], bnc)
        pltpu.sync_copy(bnc, er4_sp.at[qsl])
    plsc.subcore_barrier()

    mh = [mv[pl.ds(hh * 16, 16)] for hh in range(HPC)]

    def window(w, carry):
        base = pl.multiple_of(sid * ET6 + w * W, 8)
        pltpu.sync_copy(src_hbm.at[pl.ds(base, W)], srcv)
        pltpu.sync_copy(dst_hbm.at[pl.ds(base, W)], dstv)

        def mkidx(k, c2):
            csl = pl.ds(k * 16, 16)
            src16 = srcv[csl]
            dst16 = dstv[csl]
            for hh in range(HPC):
                idxs[hh, csl] = src16 + hh * N
                idxd[hh, csl] = dst16 + hh * N
            return c2
        lax.fori_loop(0, W // 16, mkidx, 0)
        for hh in range(HPC):
            pltpu.sync_copy(el4_sp.at[idxs.at[hh]], elb.at[hh])
            pltpu.sync_copy(er4_sp.at[idxd.at[hh]], erb.at[hh])

        def chunk(k, c2):
            csl = pl.ds(k * 16, 16)
            for hh in range(HPC):
                z = elb[hh, csl] + erb[hh, csl]
                e = jnp.maximum(z, z * 0.2)
                eeb[hh, csl] = jnp.exp(e - mh[hh])
            return c2
        lax.fori_loop(0, W // 16, chunk, 0)
        for hh in range(HPC):
            pltpu.sync_copy(eeb.at[hh], den_sp.at[idxd.at[hh]], add=True)
        return carry
    lax.fori_loop(0, NW6, window, 0)
    plsc.subcore_barrier()

    @pl.when(jnp.logical_and(cid == 0, sid < 8))
    def _():
        qsl = pl.ds(pl.multiple_of(sid * DQ, 8), DQ)
        pltpu.sync_copy(den_sp.at[qsl], bnc)
        pltpu.sync_copy(bnc, dena_hbm.at[qsl])

    @pl.when(jnp.logical_and(cid == 1, sid < 8))
    def _():
        qsl = pl.ds(pl.multiple_of(sid * DQ, 8), DQ)
        pltpu.sync_copy(den_sp.at[qsl], bnc)
        pltpu.sync_copy(bnc, denb_hbm.at[qsl])


@functools.partial(
    pl.kernel,
    out_type=jax.ShapeDtypeStruct((H, N, D), jnp.float32),
    mesh=_mesh,
    scratch_types=[
        pltpu.VMEM_SHARED((N, D), jnp.float32),
        pltpu.VMEM_SHARED((N,), jnp.float32),
        pltpu.VMEM_SHARED((N,), jnp.float32),
        pltpu.VMEM_SHARED((N,), jnp.float32),
        pltpu.VMEM((W,), jnp.int32),
        pltpu.VMEM((W,), jnp.int32),
        pltpu.VMEM((W, D), jnp.float32),
        pltpu.VMEM((W,), jnp.float32),
        pltpu.VMEM((W,), jnp.float32),
        pltpu.VMEM((W,), jnp.float32),
        pltpu.VMEM((W,), jnp.float32),
        pltpu.VMEM((16,), jnp.float32),
        pltpu.VMEM((RCH + 16,), jnp.float32),
    ],
)
def _gat_kernel(h0_hbm, elt_hbm, ert_hbm, ur_hbm, m_hbm, src_hbm, dst_hbm,
                zeros_hbm, out_hbm,
                acc_sp, el1_sp, er1_sp, ur1_sp, srcv, dstv, rows,
                elb, erb, urb, alb, mv, bnc):
    cid = lax.axis_index("c")
    sid = lax.axis_index("s")
    for p in range(HPC):
        head = cid * HPC + p
        hoff = pl.multiple_of(head * N, 8)
        pltpu.sync_copy(m_hbm.at[pl.ds(pl.multiple_of(head * 16, 8), 16)],
                        mv)
        _row_chunks(sid, lambda off, sz: pltpu.sync_copy(
            zeros_hbm.at[pl.ds(off, sz)], acc_sp.at[pl.ds(off, sz)]))

        def stage(tbl_hbm, dst_sp):
            def cp(off, sz):
                pltpu.sync_copy(tbl_hbm.at[pl.ds(hoff + off, sz)],
                                bnc.at[pl.ds(0, sz)])
                pltpu.sync_copy(bnc.at[pl.ds(0, sz)],
                                dst_sp.at[pl.ds(off, sz)])
            _row_chunks(sid, cp)
        stage(elt_hbm, el1_sp)
        stage(ert_hbm, er1_sp)
        stage(ur_hbm, ur1_sp)
        plsc.subcore_barrier()
        mhead = mv[:]

        def window(w, carry):
            base = pl.multiple_of(sid * ET6 + w * W, 8)
            pltpu.sync_copy(src_hbm.at[pl.ds(base, W)], srcv)
            pltpu.sync_copy(dst_hbm.at[pl.ds(base, W)], dstv)
            pltpu.sync_copy(h0_hbm.at[srcv], rows)
            pltpu.sync_copy(el1_sp.at[srcv], elb)
            pltpu.sync_copy(er1_sp.at[dstv], erb)
            pltpu.sync_copy(ur1_sp.at[dstv], urb)

            def chunk(k, c2):
                csl = pl.ds(k * 16, 16)
                z = elb[csl] + erb[csl]
                e = jnp.maximum(z, z * 0.2)
                alb[csl] = jnp.exp(e - mhead) * urb[csl]
                return c2
            lax.fori_loop(0, W // 16, chunk, 0)

            def scale(k, c2):
                a16 = alb[pl.ds(k * 16, 16)]
                for i in range(16):
                    a = a16[i]
                    row = k * 16 + i
                    for j in range(D // 16):
                        sl = pl.ds(j * 16, 16)
                        rows[row, sl] = rows[row, sl] * a
                return c2
            lax.fori_loop(0, W // 16, scale, 0)
            pltpu.sync_copy(rows, acc_sp.at[dstv], add=True)
            return carry
        lax.fori_loop(0, NW6, window, 0)
        plsc.subcore_barrier()
        _row_chunks(sid, lambda off, sz: pltpu.sync_copy(
            acc_sp.at[pl.ds(off, sz)], out_hbm.at[head, pl.ds(off, sz)]))
        plsc.subcore_barrier()


def _mm_body(x_ref, w_ref, o_ref):
    o_ref[...] = jnp.dot(x_ref[...], w_ref[...],
                         preferred_element_type=jnp.float32)


def _t1(x, W1):
    return pl.pallas_call(
        _mm_body,
        grid=(10,),
        in_specs=[pl.BlockSpec((N // 10, D), lambda i: (i, 0)),
                  pl.BlockSpec((D, D), lambda i: (0, 0))],
        out_specs=pl.BlockSpec((N // 10, D), lambda i: (i, 0)),
        out_shape=jax.ShapeDtypeStruct((N, D), jnp.float32),
    )(x, W1)


def _t3_body(aggA, aggB, b1, W2, al, ar, h0_o, elt_o, ert_o, m_o):
    h0 = jnp.maximum(aggA[...] + aggB[...] + b1[...], 0.0)
    h0_o[...] = h0
    w2 = W2[...]
    alv = al[...]
    arv = ar[...]
    cols_l = []
    cols_r = []
    for h in range(H):
        w2h = w2[:, h * O:(h + 1) * O]
        cols_l.append(lax.dot_general(w2h, alv[h:h + 1, :],
                                      (((1,), (1,)), ((), ()))))
        cols_r.append(lax.dot_general(w2h, arv[h:h + 1, :],
                                      (((1,), (1,)), ((), ()))))
    Wl = jnp.concatenate(cols_l, axis=1)
    Wr = jnp.concatenate(cols_r, axis=1)
    elt = lax.dot_general(Wl, h0, (((0,), (1,)), ((), ())))
    ert = lax.dot_general(Wr, h0, (((0,), (1,)), ((), ())))
    elt_o[...] = elt
    ert_o[...] = ert
    m = (jnp.max(elt, axis=1, keepdims=True)
         + jnp.max(ert, axis=1, keepdims=True))
    m_o[...] = jnp.broadcast_to(m, (H, 16))


def _t3(aggA, aggB, b1, W2, al, ar):
    return pl.pallas_call(
        _t3_body,
        out_shape=(jax.ShapeDtypeStruct((N, D), jnp.float32),
                   jax.ShapeDtypeStruct((H, N), jnp.float32),
                   jax.ShapeDtypeStruct((H, N), jnp.float32),
                   jax.ShapeDtypeStruct((H, 16), jnp.float32)),
    )(aggA, aggB, b1, W2, al, ar)


def _t5b_body(a_ref, b_ref, o_ref):
    den = jnp.concatenate([a_ref[...], b_ref[...]], axis=0)
    o_ref[...] = 1.0 / jnp.clip(den, 1e-9, None)


def _t5b(denA, denB):
    return pl.pallas_call(
        _t5b_body,
        out_shape=jax.ShapeDtypeStruct((H, N), jnp.float32),
    )(denA, denB)


def _t7_body(acc_ref, W2_ref, b2_ref, o_ref):
    w2 = W2_ref[...]
    b2 = b2_ref[...]
    acc = acc_ref[...]
    BN = acc.shape[1]
    s = jnp.zeros((BN, D), jnp.float32)
    for h in range(H):
        y = jnp.dot(acc[h], w2[:, h * O:(h + 1) * O],
                    preferred_element_type=jnp.float32) + b2[:, h * O:(h + 1) * O]
        s = s + jnp.maximum(y, 0.0)
    o_ref[...] = s * (1.0 / H)


def _t7(acc8, W2, b2):
    BN = N // 10
    return pl.pallas_call(
        _t7_body,
        grid=(10,),
        in_specs=[pl.BlockSpec((H, BN, D), lambda i: (0, i, 0)),
                  pl.BlockSpec((D, H * O), lambda i: (0, 0)),
                  pl.BlockSpec((1, H * O), lambda i: (0, 0))],
        out_specs=pl.BlockSpec((BN, D), lambda i: (i, 0)),
        out_shape=jax.ShapeDtypeStruct((N, D), jnp.float32),
    )(acc8, W2, b2)


def kernel(x, edge_index, edge_weight, W1, b1, W2, attn_l, attn_r, b2):
    src = edge_index[0]
    dst = edge_index[1]
    zeros_nd = jnp.zeros((N, D), jnp.float32)
    zeros_q = jnp.zeros((HPC * N,), jnp.float32)

    h = _t1(x, W1)
    aggA, aggB = _agg0_kernel(h, src, dst, edge_weight, zeros_nd)
    h0, elt, ert, mb = _t3(aggA, aggB, b1.reshape(1, D), W2,
                           attn_l, attn_r)
    m16 = mb.reshape(H * 16)
    elt_f = elt.reshape(H * N)
    ert_f = ert.reshape(H * N)
    denA, denB = _den_kernel(elt_f, ert_f, m16, src, dst, zeros_q)
    urec = _t5b(denA.reshape(HPC, N), denB.reshape(HPC, N))
    acc8 = _gat_kernel(h0, elt_f, ert_f, urec.reshape(H * N), m16,
                       src, dst, zeros_nd)
    out = _t7(acc8, W2, b2.reshape(1, H * O))
    return out


# speedup vs baseline: 20.2631x; 2.0447x over previous
import functools

import jax
import jax.numpy as jnp
from jax import lax
from jax.experimental import pallas as pl
from jax.experimental.pallas import tpu as pltpu
from jax.experimental.pallas import tpu_sc as plsc

N = 10000
E = 320000
D = 128
H = 8
O = 128

NC, NS = 2, 16
W = 80
ET2 = E // (NC * NS)
NW2 = ET2 // W
ET6 = E // NS
NW6 = ET6 // W
RCH = 624
HPC = H // NC
DQ = (HPC * N) // 8

_mesh = plsc.VectorSubcoreMesh(core_axis_name="c", subcore_axis_name="s")


def _row_chunks(sid, fn):
    off = pl.multiple_of(sid * RCH, 8)
    fn(off, RCH)

    @pl.when(sid == NS - 1)
    def _():
        fn(NS * RCH, N - NS * RCH)


NW2H = NW2 // 2


@functools.partial(
    pl.kernel,
    out_type=(jax.ShapeDtypeStruct((N, D), jnp.float32),
              jax.ShapeDtypeStruct((N, D), jnp.float32)),
    mesh=_mesh,
    scratch_types=[
        pltpu.VMEM_SHARED((N, D), jnp.float32),
        pltpu.VMEM((2, W), jnp.int32),
        pltpu.VMEM((2, W), jnp.int32),
        pltpu.VMEM((W,), jnp.int32),
        pltpu.VMEM((W,), jnp.int32),
        pltpu.VMEM((W,), jnp.float32),
        pltpu.VMEM((W,), jnp.float32),
        pltpu.VMEM((W, D), jnp.float32),
        pltpu.VMEM((W, D), jnp.float32),
        pltpu.SemaphoreType.DMA,
        pltpu.SemaphoreType.DMA,
        pltpu.SemaphoreType.DMA,
        pltpu.SemaphoreType.DMA,
        pltpu.SemaphoreType.DMA,
        pltpu.SemaphoreType.DMA,
    ],
)
def _agg0_kernel(h_hbm, src_hbm, dst_hbm, ew_hbm, zeros_hbm,
                 outa_hbm, outb_hbm, acc_sp,
                 idxA, idxB, sidxA, sidxB, ewA, ewB, rowsA, rowsB,
                 isemA, isemB, grA, grB, ssA, ssB):
    cid = lax.axis_index("c")
    sid = lax.axis_index("s")
    wid = cid * NS + sid
    _row_chunks(sid, lambda off, sz: pltpu.sync_copy(
        zeros_hbm.at[pl.ds(off, sz)], acc_sp.at[pl.ds(off, sz)]))
    plsc.subcore_barrier()

    def win_base(w):
        return pl.multiple_of(wid * ET2 + w * W, 8)

    def fetch(wi, buf, ewb, sem):
        b = win_base(wi)
        pltpu.async_copy(src_hbm.at[pl.ds(b, W)], buf.at[0], sem)
        pltpu.async_copy(dst_hbm.at[pl.ds(b, W)], buf.at[1], sem)
        pltpu.async_copy(ew_hbm.at[pl.ds(b, W)], ewb, sem)

    def wait_fetch(buf, ewb, sem):
        pltpu.make_async_copy(
            src_hbm.at[pl.ds(0, W)], buf.at[0], sem).wait()
        pltpu.make_async_copy(
            dst_hbm.at[pl.ds(0, W)], buf.at[1], sem).wait()
        pltpu.make_async_copy(ew_hbm.at[pl.ds(0, W)], ewb, sem).wait()

    def compute(idx2, ewb, rows, sidx):
        def scalef(k, c2):
            csl = pl.ds(k * 16, 16)
            sidx[csl] = idx2[1, csl]
            w16 = ewb[csl]
            for i in range(16):
                wgt = w16[i]
                row = k * 16 + i
                for j in range(D // 16):
                    sl = pl.ds(j * 16, 16)
                    rows[row, sl] = rows[row, sl] * wgt
            return c2
        lax.fori_loop(0, W // 16, scalef, 0)

    fetch(0, idxA, ewA, isemA)

    def body(w2, carry):
        e2 = w2 * 2
        wait_fetch(idxA, ewA, isemA)

        @pl.when(w2 > 0)
        def _():
            pltpu.make_async_copy(rowsA, acc_sp.at[sidxA], ssA).wait()
        ga = pltpu.async_copy(h_hbm.at[idxA.at[0]], rowsA, grA)
        fetch(e2 + 1, idxB, ewB, isemB)
        ga.wait()
        compute(idxA, ewA, rowsA, sidxA)
        pltpu.async_copy(rowsA, acc_sp.at[sidxA], ssA, add=True)
        wait_fetch(idxB, ewB, isemB)

        @pl.when(w2 > 0)
        def _():
            pltpu.make_async_copy(rowsB, acc_sp.at[sidxB], ssB).wait()
        gb = pltpu.async_copy(h_hbm.at[idxB.at[0]], rowsB, grB)
        fetch(e2 + 2, idxA, ewA, isemA)
        gb.wait()
        compute(idxB, ewB, rowsB, sidxB)
        pltpu.async_copy(rowsB, acc_sp.at[sidxB], ssB, add=True)
        return carry
    lax.fori_loop(0, NW2H, body, 0)
    wait_fetch(idxA, ewA, isemA)
    pltpu.make_async_copy(rowsA, acc_sp.at[sidxA], ssA).wait()
    ga = pltpu.async_copy(h_hbm.at[idxA.at[0]], rowsA, grA)
    ga.wait()
    compute(idxA, ewA, rowsA, sidxA)
    pltpu.async_copy(rowsA, acc_sp.at[sidxA], ssA, add=True)
    pltpu.make_async_copy(rowsA, acc_sp.at[sidxA], ssA).wait()
    pltpu.make_async_copy(rowsB, acc_sp.at[sidxB], ssB).wait()
    plsc.subcore_barrier()

    @pl.when(cid == 0)
    def _():
        _row_chunks(sid, lambda off, sz: pltpu.sync_copy(
            acc_sp.at[pl.ds(off, sz)], outa_hbm.at[pl.ds(off, sz)]))

    @pl.when(cid == 1)
    def _():
        _row_chunks(sid, lambda off, sz: pltpu.sync_copy(
            acc_sp.at[pl.ds(off, sz)], outb_hbm.at[pl.ds(off, sz)]))


@functools.partial(
    pl.kernel,
    out_type=(jax.ShapeDtypeStruct((HPC * N,), jnp.float32),
              jax.ShapeDtypeStruct((HPC * N,), jnp.float32)),
    mesh=_mesh,
    scratch_types=[
        pltpu.VMEM_SHARED((HPC * N,), jnp.float32),
        pltpu.VMEM_SHARED((HPC * N,), jnp.float32),
        pltpu.VMEM_SHARED((HPC * N,), jnp.float32),
        pltpu.VMEM((2, W), jnp.int32),
        pltpu.VMEM((HPC, W), jnp.int32),
        pltpu.VMEM((HPC, W), jnp.int32),
        pltpu.VMEM((HPC, W), jnp.float32),
        pltpu.VMEM((HPC, W), jnp.float32),
        pltpu.VMEM((HPC, W), jnp.float32),
        pltpu.VMEM((HPC * 16,), jnp.float32),
        pltpu.VMEM((DQ,), jnp.float32),
        pltpu.SemaphoreType.DMA,
        pltpu.SemaphoreType.DMA,
        pltpu.SemaphoreType.DMA,
    ],
)
def _den_kernel(elt_hbm, ert_hbm, m_hbm, src_hbm, dst_hbm, zeros_hbm,
                dena_hbm, denb_hbm,
                den_sp, el4_sp, er4_sp, srcv, idxs, idxd,
                elb, erb, eeb, mv, bnc, gsem1, gsem2, ssem):
    cid = lax.axis_index("c")
    sid = lax.axis_index("s")
    toff = pl.multiple_of(cid * (HPC * N), 8)
    moff = pl.multiple_of(cid * (HPC * 16), 8)
    pltpu.sync_copy(m_hbm.at[pl.ds(moff, HPC * 16)], mv)

    @pl.when(sid < 8)
    def _():
        qsl = pl.ds(pl.multiple_of(sid * DQ, 8), DQ)
        tsl = pl.ds(pl.multiple_of(toff + sid * DQ, 8), DQ)
        pltpu.sync_copy(zeros_hbm.at[qsl], bnc)
        pltpu.sync_copy(bnc, den_sp.at[qsl])
        pltpu.sync_copy(elt_hbm.at[tsl], bnc)
        pltpu.sync_copy(bnc, el4_sp.at[qsl])
        pltpu.sync_copy(ert_hbm.at[tsl], bnc)
        pltpu.sync_copy(bnc, er4_sp.at[qsl])
    plsc.subcore_barrier()

    mh = [mv[pl.ds(hh * 16, 16)] for hh in range(HPC)]

    def window(w, carry):
        base = pl.multiple_of(sid * ET6 + w * W, 8)
        i1 = pltpu.async_copy(src_hbm.at[pl.ds(base, W)], srcv.at[0],
                              gsem1)
        i2 = pltpu.async_copy(dst_hbm.at[pl.ds(base, W)], srcv.at[1],
                              gsem2)
        i1.wait()
        i2.wait()
        @pl.when(w > 0)
        def _():
            for hh in range(HPC):
                pltpu.make_async_copy(
                    eeb.at[hh], den_sp.at[idxd.at[hh]], ssem).wait()

        def mkidx(k, c2):
            csl = pl.ds(k * 16, 16)
            src16 = srcv[0, csl]
            dst16 = srcv[1, csl]
            for hh in range(HPC):
                idxs[hh, csl] = src16 + hh * N
                idxd[hh, csl] = dst16 + hh * N
            return c2
        lax.fori_loop(0, W // 16, mkidx, 0)
        gs = []
        for hh in range(HPC):
            gs.append(pltpu.async_copy(
                el4_sp.at[idxs.at[hh]], elb.at[hh], gsem1))
            gs.append(pltpu.async_copy(
                er4_sp.at[idxd.at[hh]], erb.at[hh], gsem2))
        for g in gs:
            g.wait()

        def chunk(k, c2):
            csl = pl.ds(k * 16, 16)
            for hh in range(HPC):
                z = elb[hh, csl] + erb[hh, csl]
                e = jnp.maximum(z, z * 0.2)
                eeb[hh, csl] = jnp.exp(e - mh[hh])
            return c2
        lax.fori_loop(0, W // 16, chunk, 0)
        for hh in range(HPC):
            pltpu.async_copy(eeb.at[hh], den_sp.at[idxd.at[hh]], ssem,
                             add=True)
        return carry
    lax.fori_loop(0, NW6, window, 0)
    for hh in range(HPC):
        pltpu.make_async_copy(eeb.at[hh], den_sp.at[idxd.at[hh]],
                              ssem).wait()
    plsc.subcore_barrier()

    @pl.when(jnp.logical_and(cid == 0, sid < 8))
    def _():
        qsl = pl.ds(pl.multiple_of(sid * DQ, 8), DQ)
        pltpu.sync_copy(den_sp.at[qsl], bnc)
        pltpu.sync_copy(bnc, dena_hbm.at[qsl])

    @pl.when(jnp.logical_and(cid == 1, sid < 8))
    def _():
        qsl = pl.ds(pl.multiple_of(sid * DQ, 8), DQ)
        pltpu.sync_copy(den_sp.at[qsl], bnc)
        pltpu.sync_copy(bnc, denb_hbm.at[qsl])


NWH = NW6 // 2


@functools.partial(
    pl.kernel,
    out_type=jax.ShapeDtypeStruct((H, N, D), jnp.float32),
    mesh=_mesh,
    scratch_types=[
        pltpu.VMEM_SHARED((N, D), jnp.float32),
        pltpu.VMEM_SHARED((N,), jnp.float32),
        pltpu.VMEM_SHARED((N,), jnp.float32),
        pltpu.VMEM((2, W), jnp.int32),
        pltpu.VMEM((2, W), jnp.int32),
        pltpu.VMEM((W,), jnp.int32),
        pltpu.VMEM((W,), jnp.int32),
        pltpu.VMEM((W, D), jnp.float32),
        pltpu.VMEM((W, D), jnp.float32),
        pltpu.VMEM((W,), jnp.float32),
        pltpu.VMEM((W,), jnp.float32),
        pltpu.VMEM((W,), jnp.float32),
        pltpu.VMEM((W,), jnp.float32),
        pltpu.VMEM((W,), jnp.float32),
        pltpu.VMEM((16,), jnp.float32),
        pltpu.VMEM((RCH + 16,), jnp.float32),
        pltpu.SemaphoreType.DMA,
        pltpu.SemaphoreType.DMA,
        pltpu.SemaphoreType.DMA,
        pltpu.SemaphoreType.DMA,
        pltpu.SemaphoreType.DMA,
        pltpu.SemaphoreType.DMA,
        pltpu.SemaphoreType.DMA,
        pltpu.SemaphoreType.DMA,
        pltpu.SemaphoreType.DMA,
        pltpu.SemaphoreType.DMA,
    ],
)
def _gat_kernel(h0_hbm, elt_hbm, ert_hbm, m_hbm, src_hbm, dst_hbm,
                zeros_hbm, out_hbm,
                acc_sp, el1_sp, er1_sp,
                idxA, idxB, sidxA, sidxB, rowsA, rowsB,
                elbA, elbB, erbA, erbB, alb, mv, bnc,
                isemA, isemB, grA, grB, geA, geB, gfA, gfB, ssA, ssB):
    cid = lax.axis_index("c")
    sid = lax.axis_index("s")

    def win_base(w):
        return pl.multiple_of(sid * ET6 + w * W, 8)

    def compute_window(idx2, elb, erb, rows, sidx, mhead):
        def chunkf(k, c2):
            csl = pl.ds(k * 16, 16)
            z = elb[csl] + erb[csl]
            e = jnp.maximum(z, z * 0.2)
            alb[csl] = jnp.exp(e - mhead)
            sidx[csl] = idx2[1, csl]
            return c2
        lax.fori_loop(0, W // 16, chunkf, 0)

        def scalef(k, c2):
            a16 = alb[pl.ds(k * 16, 16)]
            for i in range(16):
                a = a16[i]
                row = k * 16 + i
                for j in range(D // 16):
                    sl = pl.ds(j * 16, 16)
                    rows[row, sl] = rows[row, sl] * a
            return c2
        lax.fori_loop(0, W // 16, scalef, 0)

    for p in range(HPC):
        head = cid * HPC + p
        hoff = pl.multiple_of(head * N, 8)
        pltpu.sync_copy(m_hbm.at[pl.ds(pl.multiple_of(head * 16, 8), 16)],
                        mv)
        _row_chunks(sid, lambda off, sz: pltpu.sync_copy(
            zeros_hbm.at[pl.ds(off, sz)], acc_sp.at[pl.ds(off, sz)]))

        def stage(tbl_hbm, dst_sp):
            def cp(off, sz):
                pltpu.sync_copy(tbl_hbm.at[pl.ds(hoff + off, sz)],
                                bnc.at[pl.ds(0, sz)])
                pltpu.sync_copy(bnc.at[pl.ds(0, sz)],
                                dst_sp.at[pl.ds(off, sz)])
            _row_chunks(sid, cp)
        stage(elt_hbm, el1_sp)
        stage(ert_hbm, er1_sp)
        plsc.subcore_barrier()
        mhead = mv[:]

        def fetch_idx(wi, buf, sem):
            b = win_base(wi)
            pltpu.async_copy(src_hbm.at[pl.ds(b, W)], buf.at[0], sem)
            pltpu.async_copy(dst_hbm.at[pl.ds(b, W)], buf.at[1], sem)

        def wait_idx(buf, sem):
            pltpu.make_async_copy(
                src_hbm.at[pl.ds(0, W)], buf.at[0], sem).wait()
            pltpu.make_async_copy(
                dst_hbm.at[pl.ds(0, W)], buf.at[1], sem).wait()

        fetch_idx(0, idxA, isemA)

        def body(w2, carry):
            e2 = w2 * 2
            wait_idx(idxA, isemA)

            @pl.when(w2 > 0)
            def _():
                pltpu.make_async_copy(
                    rowsA, acc_sp.at[sidxA], ssA).wait()
            ga = pltpu.async_copy(h0_hbm.at[idxA.at[0]], rowsA, grA)
            gb = pltpu.async_copy(el1_sp.at[idxA.at[0]], elbA, geA)
            gc = pltpu.async_copy(er1_sp.at[idxA.at[1]], erbA, gfA)
            fetch_idx(e2 + 1, idxB, isemB)
            ga.wait()
            gb.wait()
            gc.wait()
            compute_window(idxA, elbA, erbA, rowsA, sidxA, mhead)
            pltpu.async_copy(rowsA, acc_sp.at[sidxA], ssA, add=True)
            wait_idx(idxB, isemB)

            @pl.when(w2 > 0)
            def _():
                pltpu.make_async_copy(
                    rowsB, acc_sp.at[sidxB], ssB).wait()
            ga2 = pltpu.async_copy(h0_hbm.at[idxB.at[0]], rowsB, grB)
            gb2 = pltpu.async_copy(el1_sp.at[idxB.at[0]], elbB, geB)
            gc2 = pltpu.async_copy(er1_sp.at[idxB.at[1]], erbB, gfB)

            @pl.when(w2 < NWH - 1)
            def _():
                fetch_idx(e2 + 2, idxA, isemA)
            ga2.wait()
            gb2.wait()
            gc2.wait()
            compute_window(idxB, elbB, erbB, rowsB, sidxB, mhead)
            pltpu.async_copy(rowsB, acc_sp.at[sidxB], ssB, add=True)
            return carry
        lax.fori_loop(0, NWH, body, 0)
        pltpu.make_async_copy(rowsA, acc_sp.at[sidxA], ssA).wait()
        pltpu.make_async_copy(rowsB, acc_sp.at[sidxB], ssB).wait()
        plsc.subcore_barrier()
        _row_chunks(sid, lambda off, sz: pltpu.sync_copy(
            acc_sp.at[pl.ds(off, sz)], out_hbm.at[head, pl.ds(off, sz)]))
        plsc.subcore_barrier()


def _mm_body(x_ref, w_ref, o_ref):
    o_ref[...] = jnp.dot(x_ref[...], w_ref[...],
                         preferred_element_type=jnp.float32)


def _t1(x, W1):
    return pl.pallas_call(
        _mm_body,
        grid=(10,),
        in_specs=[pl.BlockSpec((N // 10, D), lambda i: (i, 0)),
                  pl.BlockSpec((D, D), lambda i: (0, 0))],
        out_specs=pl.BlockSpec((N // 10, D), lambda i: (i, 0)),
        out_shape=jax.ShapeDtypeStruct((N, D), jnp.float32),
    )(x, W1)


def _t3_body(aggA, aggB, b1, W2, al, ar, h0_o, elt_o, ert_o, m_o):
    h0 = jnp.maximum(aggA[...] + aggB[...] + b1[...], 0.0)
    h0_o[...] = h0
    w2 = W2[...]
    alv = al[...]
    arv = ar[...]
    cols_l = []
    cols_r = []
    for h in range(H):
        w2h = w2[:, h * O:(h + 1) * O]
        cols_l.append(lax.dot_general(w2h, alv[h:h + 1, :],
                                      (((1,), (1,)), ((), ()))))
        cols_r.append(lax.dot_general(w2h, arv[h:h + 1, :],
                                      (((1,), (1,)), ((), ()))))
    Wl = jnp.concatenate(cols_l, axis=1)
    Wr = jnp.concatenate(cols_r, axis=1)
    elt = lax.dot_general(Wl, h0, (((0,), (1,)), ((), ())))
    ert = lax.dot_general(Wr, h0, (((0,), (1,)), ((), ())))
    elt_o[...] = elt
    ert_o[...] = ert
    m = (jnp.max(elt, axis=1, keepdims=True)
         + jnp.max(ert, axis=1, keepdims=True))
    m_o[...] = jnp.broadcast_to(m, (H, 16))


def _t3(aggA, aggB, b1, W2, al, ar):
    return pl.pallas_call(
        _t3_body,
        out_shape=(jax.ShapeDtypeStruct((N, D), jnp.float32),
                   jax.ShapeDtypeStruct((H, N), jnp.float32),
                   jax.ShapeDtypeStruct((H, N), jnp.float32),
                   jax.ShapeDtypeStruct((H, 16), jnp.float32)),
    )(aggA, aggB, b1, W2, al, ar)


def _t5b_body(a_ref, b_ref, o_ref):
    den = jnp.concatenate([a_ref[...], b_ref[...]], axis=0)
    o_ref[...] = 1.0 / jnp.clip(den, 1e-9, None)


def _t5b(denA, denB):
    return pl.pallas_call(
        _t5b_body,
        out_shape=jax.ShapeDtypeStruct((H, N), jnp.float32),
    )(denA, denB)


def _t7_body(acc_ref, ur_ref, W2_ref, b2_ref, o_ref):
    w2 = W2_ref[...]
    b2 = b2_ref[...]
    acc = acc_ref[...]
    ur = ur_ref[...]
    BN = acc.shape[1]
    s = jnp.zeros((BN, D), jnp.float32)
    for h in range(H):
        y = jnp.dot(acc[h], w2[:, h * O:(h + 1) * O],
                    preferred_element_type=jnp.float32)
        y = y * ur[:, h:h + 1] + b2[:, h * O:(h + 1) * O]
        s = s + jnp.maximum(y, 0.0)
    o_ref[...] = s * (1.0 / H)


def _t7(acc8, urec, W2, b2):
    BN = N // 10
    return pl.pallas_call(
        _t7_body,
        grid=(10,),
        in_specs=[pl.BlockSpec((H, BN, D), lambda i: (0, i, 0)),
                  pl.BlockSpec((BN, H), lambda i: (i, 0)),
                  pl.BlockSpec((D, H * O), lambda i: (0, 0)),
                  pl.BlockSpec((1, H * O), lambda i: (0, 0))],
        out_specs=pl.BlockSpec((BN, D), lambda i: (i, 0)),
        out_shape=jax.ShapeDtypeStruct((N, D), jnp.float32),
    )(acc8, urec, W2, b2)


def kernel(x, edge_index, edge_weight, W1, b1, W2, attn_l, attn_r, b2):
    src = edge_index[0]
    dst = edge_index[1]
    zeros_nd = jnp.zeros((N, D), jnp.float32)
    zeros_q = jnp.zeros((HPC * N,), jnp.float32)

    h = _t1(x, W1)
    aggA, aggB = _agg0_kernel(h, src, dst, edge_weight, zeros_nd)
    h0, elt, ert, mb = _t3(aggA, aggB, b1.reshape(1, D), W2,
                           attn_l, attn_r)
    m16 = mb.reshape(H * 16)
    elt_f = elt.reshape(H * N)
    ert_f = ert.reshape(H * N)
    denA, denB = _den_kernel(elt_f, ert_f, m16, src, dst, zeros_q)
    urec = _t5b(denA.reshape(HPC, N), denB.reshape(HPC, N))
    acc8 = _gat_kernel(h0, elt_f, ert_f, m16, src, dst, zeros_nd)
    out = _t7(acc8, urec.T, W2, b2.reshape(1, H * O))
    return out


# speedup vs baseline: 22.4422x; 1.1075x over previous
import functools

import jax
import jax.numpy as jnp
from jax import lax
from jax.experimental import pallas as pl
from jax.experimental.pallas import tpu as pltpu
from jax.experimental.pallas import tpu_sc as plsc

N = 10000
E = 320000
D = 128
H = 8
O = 128

NC, NS = 2, 16
W = 80
ET2 = E // (NC * NS)
NW2 = ET2 // W
ET6 = E // NS
NW6 = ET6 // W
RCH = 624
HPC = H // NC
DQ = (HPC * N) // 8

_mesh = plsc.VectorSubcoreMesh(core_axis_name="c", subcore_axis_name="s")


def _row_chunks(sid, fn):
    off = pl.multiple_of(sid * RCH, 8)
    fn(off, RCH)

    @pl.when(sid == NS - 1)
    def _():
        fn(NS * RCH, N - NS * RCH)


NW2H = NW2 // 2


@functools.partial(
    pl.kernel,
    out_type=(jax.ShapeDtypeStruct((N, D), jnp.float32),
              jax.ShapeDtypeStruct((N, D), jnp.float32)),
    mesh=_mesh,
    scratch_types=[
        pltpu.VMEM_SHARED((N, D), jnp.float32),
        pltpu.VMEM((2, W), jnp.int32),
        pltpu.VMEM((2, W), jnp.int32),
        pltpu.VMEM((W,), jnp.int32),
        pltpu.VMEM((W,), jnp.int32),
        pltpu.VMEM((W,), jnp.float32),
        pltpu.VMEM((W,), jnp.float32),
        pltpu.VMEM((W, D), jnp.float32),
        pltpu.VMEM((W, D), jnp.float32),
        pltpu.SemaphoreType.DMA,
        pltpu.SemaphoreType.DMA,
        pltpu.SemaphoreType.DMA,
        pltpu.SemaphoreType.DMA,
        pltpu.SemaphoreType.DMA,
        pltpu.SemaphoreType.DMA,
    ],
)
def _agg0_kernel(h_hbm, src_hbm, dst_hbm, ew_hbm, zeros_hbm,
                 outa_hbm, outb_hbm, acc_sp,
                 idxA, idxB, sidxA, sidxB, ewA, ewB, rowsA, rowsB,
                 isemA, isemB, grA, grB, ssA, ssB):
    cid = lax.axis_index("c")
    sid = lax.axis_index("s")
    wid = cid * NS + sid
    _row_chunks(sid, lambda off, sz: pltpu.sync_copy(
        zeros_hbm.at[pl.ds(off, sz)], acc_sp.at[pl.ds(off, sz)]))
    plsc.subcore_barrier()

    def win_base(w):
        return pl.multiple_of(wid * ET2 + w * W, 8)

    def fetch(wi, buf, ewb, sem):
        b = win_base(wi)
        pltpu.async_copy(src_hbm.at[pl.ds(b, W)], buf.at[0], sem)
        pltpu.async_copy(dst_hbm.at[pl.ds(b, W)], buf.at[1], sem)
        pltpu.async_copy(ew_hbm.at[pl.ds(b, W)], ewb, sem)

    def wait_fetch(buf, ewb, sem):
        pltpu.make_async_copy(
            src_hbm.at[pl.ds(0, W)], buf.at[0], sem).wait()
        pltpu.make_async_copy(
            dst_hbm.at[pl.ds(0, W)], buf.at[1], sem).wait()
        pltpu.make_async_copy(ew_hbm.at[pl.ds(0, W)], ewb, sem).wait()

    def compute(idx2, ewb, rows, sidx):
        def scalef(k, c2):
            csl = pl.ds(k * 16, 16)
            sidx[csl] = idx2[1, csl]
            w16 = ewb[csl]
            for i in range(16):
                wgt = w16[i]
                row = k * 16 + i
                for j in range(D // 16):
                    sl = pl.ds(j * 16, 16)
                    rows[row, sl] = rows[row, sl] * wgt
            return c2
        lax.fori_loop(0, W // 16, scalef, 0)

    fetch(0, idxA, ewA, isemA)

    def body(w2, carry):
        e2 = w2 * 2
        wait_fetch(idxA, ewA, isemA)

        @pl.when(w2 > 0)
        def _():
            pltpu.make_async_copy(rowsA, acc_sp.at[sidxA], ssA).wait()
        ga = pltpu.async_copy(h_hbm.at[idxA.at[0]], rowsA, grA)
        fetch(e2 + 1, idxB, ewB, isemB)
        ga.wait()
        compute(idxA, ewA, rowsA, sidxA)
        pltpu.async_copy(rowsA, acc_sp.at[sidxA], ssA, add=True)
        wait_fetch(idxB, ewB, isemB)

        @pl.when(w2 > 0)
        def _():
            pltpu.make_async_copy(rowsB, acc_sp.at[sidxB], ssB).wait()
        gb = pltpu.async_copy(h_hbm.at[idxB.at[0]], rowsB, grB)
        fetch(e2 + 2, idxA, ewA, isemA)
        gb.wait()
        compute(idxB, ewB, rowsB, sidxB)
        pltpu.async_copy(rowsB, acc_sp.at[sidxB], ssB, add=True)
        return carry
    lax.fori_loop(0, NW2H, body, 0)
    wait_fetch(idxA, ewA, isemA)
    pltpu.make_async_copy(rowsA, acc_sp.at[sidxA], ssA).wait()
    ga = pltpu.async_copy(h_hbm.at[idxA.at[0]], rowsA, grA)
    ga.wait()
    compute(idxA, ewA, rowsA, sidxA)
    pltpu.async_copy(rowsA, acc_sp.at[sidxA], ssA, add=True)
    pltpu.make_async_copy(rowsA, acc_sp.at[sidxA], ssA).wait()
    pltpu.make_async_copy(rowsB, acc_sp.at[sidxB], ssB).wait()
    plsc.subcore_barrier()

    @pl.when(cid == 0)
    def _():
        _row_chunks(sid, lambda off, sz: pltpu.sync_copy(
            acc_sp.at[pl.ds(off, sz)], outa_hbm.at[pl.ds(off, sz)]))

    @pl.when(cid == 1)
    def _():
        _row_chunks(sid, lambda off, sz: pltpu.sync_copy(
            acc_sp.at[pl.ds(off, sz)], outb_hbm.at[pl.ds(off, sz)]))


NWH = NW6 // 2


@functools.partial(
    pl.kernel,
    out_type=(jax.ShapeDtypeStruct((H, N, D), jnp.float32),
              jax.ShapeDtypeStruct((H * N,), jnp.float32)),
    mesh=_mesh,
    scratch_types=[
        pltpu.VMEM_SHARED((N, D), jnp.float32),
        pltpu.VMEM_SHARED((N,), jnp.float32),
        pltpu.VMEM_SHARED((N,), jnp.float32),
        pltpu.VMEM_SHARED((N,), jnp.float32),
        pltpu.VMEM((2, W), jnp.int32),
        pltpu.VMEM((2, W), jnp.int32),
        pltpu.VMEM((W,), jnp.int32),
        pltpu.VMEM((W,), jnp.int32),
        pltpu.VMEM((W, D), jnp.float32),
        pltpu.VMEM((W, D), jnp.float32),
        pltpu.VMEM((W,), jnp.float32),
        pltpu.VMEM((W,), jnp.float32),
        pltpu.VMEM((W,), jnp.float32),
        pltpu.VMEM((W,), jnp.float32),
        pltpu.VMEM((W,), jnp.float32),
        pltpu.VMEM((W,), jnp.float32),
        pltpu.VMEM((16,), jnp.float32),
        pltpu.VMEM((RCH + 16,), jnp.float32),
        pltpu.SemaphoreType.DMA,
        pltpu.SemaphoreType.DMA,
        pltpu.SemaphoreType.DMA,
        pltpu.SemaphoreType.DMA,
        pltpu.SemaphoreType.DMA,
        pltpu.SemaphoreType.DMA,
        pltpu.SemaphoreType.DMA,
        pltpu.SemaphoreType.DMA,
        pltpu.SemaphoreType.DMA,
        pltpu.SemaphoreType.DMA,
        pltpu.SemaphoreType.DMA,
        pltpu.SemaphoreType.DMA,
    ],
)
def _gat_kernel(h0_hbm, elt_hbm, ert_hbm, m_hbm, src_hbm, dst_hbm,
                zeros_hbm, zerosn_hbm, out_hbm, deno_hbm,
                acc_sp, el1_sp, er1_sp, den_sp,
                idxA, idxB, sidxA, sidxB, rowsA, rowsB,
                elbA, elbB, erbA, erbB, albA, albB, mv, bnc,
                isemA, isemB, grA, grB, geA, geB, gfA, gfB, ssA, ssB,
                sdA, sdB):
    cid = lax.axis_index("c")
    sid = lax.axis_index("s")

    def win_base(w):
        return pl.multiple_of(sid * ET6 + w * W, 8)

    def compute_window(idx2, elb, erb, rows, sidx, alb, mhead):
        def chunkf(k, c2):
            csl = pl.ds(k * 16, 16)
            z = elb[csl] + erb[csl]
            e = jnp.maximum(z, z * 0.2)
            alb[csl] = jnp.exp(e - mhead)
            sidx[csl] = idx2[1, csl]
            return c2
        lax.fori_loop(0, W // 16, chunkf, 0)

        def scalef(k, c2):
            a16 = alb[pl.ds(k * 16, 16)]
            for i in range(16):
                a = a16[i]
                row = k * 16 + i
                for j in range(D // 16):
                    sl = pl.ds(j * 16, 16)
                    rows[row, sl] = rows[row, sl] * a
            return c2
        lax.fori_loop(0, W // 16, scalef, 0)

    for p in range(HPC):
        head = cid * HPC + p
        hoff = pl.multiple_of(head * N, 8)
        pltpu.sync_copy(m_hbm.at[pl.ds(pl.multiple_of(head * 16, 8), 16)],
                        mv)
        _row_chunks(sid, lambda off, sz: pltpu.sync_copy(
            zeros_hbm.at[pl.ds(off, sz)], acc_sp.at[pl.ds(off, sz)]))

        def zden(off, sz):
            pltpu.sync_copy(zerosn_hbm.at[pl.ds(off, sz)],
                            bnc.at[pl.ds(0, sz)])
            pltpu.sync_copy(bnc.at[pl.ds(0, sz)],
                            den_sp.at[pl.ds(off, sz)])
        _row_chunks(sid, zden)

        def stage(tbl_hbm, dst_sp):
            def cp(off, sz):
                pltpu.sync_copy(tbl_hbm.at[pl.ds(hoff + off, sz)],
                                bnc.at[pl.ds(0, sz)])
                pltpu.sync_copy(bnc.at[pl.ds(0, sz)],
                                dst_sp.at[pl.ds(off, sz)])
            _row_chunks(sid, cp)
        stage(elt_hbm, el1_sp)
        stage(ert_hbm, er1_sp)
        plsc.subcore_barrier()
        mhead = mv[:]

        def fetch_idx(wi, buf, sem):
            b = win_base(wi)
            pltpu.async_copy(src_hbm.at[pl.ds(b, W)], buf.at[0], sem)
            pltpu.async_copy(dst_hbm.at[pl.ds(b, W)], buf.at[1], sem)

        def wait_idx(buf, sem):
            pltpu.make_async_copy(
                src_hbm.at[pl.ds(0, W)], buf.at[0], sem).wait()
            pltpu.make_async_copy(
                dst_hbm.at[pl.ds(0, W)], buf.at[1], sem).wait()

        fetch_idx(0, idxA, isemA)

        def body(w2, carry):
            e2 = w2 * 2
            wait_idx(idxA, isemA)

            @pl.when(w2 > 0)
            def _():
                pltpu.make_async_copy(
                    rowsA, acc_sp.at[sidxA], ssA).wait()
                pltpu.make_async_copy(
                    albA, den_sp.at[sidxA], sdA).wait()
            ga = pltpu.async_copy(h0_hbm.at[idxA.at[0]], rowsA, grA)
            gb = pltpu.async_copy(el1_sp.at[idxA.at[0]], elbA, geA)
            gc = pltpu.async_copy(er1_sp.at[idxA.at[1]], erbA, gfA)
            fetch_idx(e2 + 1, idxB, isemB)
            ga.wait()
            gb.wait()
            gc.wait()
            compute_window(idxA, elbA, erbA, rowsA, sidxA, albA, mhead)
            pltpu.async_copy(rowsA, acc_sp.at[sidxA], ssA, add=True)
            pltpu.async_copy(albA, den_sp.at[sidxA], sdA, add=True)
            wait_idx(idxB, isemB)

            @pl.when(w2 > 0)
            def _():
                pltpu.make_async_copy(
                    rowsB, acc_sp.at[sidxB], ssB).wait()
                pltpu.make_async_copy(
                    albB, den_sp.at[sidxB], sdB).wait()
            ga2 = pltpu.async_copy(h0_hbm.at[idxB.at[0]], rowsB, grB)
            gb2 = pltpu.async_copy(el1_sp.at[idxB.at[0]], elbB, geB)
            gc2 = pltpu.async_copy(er1_sp.at[idxB.at[1]], erbB, gfB)

            @pl.when(w2 < NWH - 1)
            def _():
                fetch_idx(e2 + 2, idxA, isemA)
            ga2.wait()
            gb2.wait()
            gc2.wait()
            compute_window(idxB, elbB, erbB, rowsB, sidxB, albB, mhead)
            pltpu.async_copy(rowsB, acc_sp.at[sidxB], ssB, add=True)
            pltpu.async_copy(albB, den_sp.at[sidxB], sdB, add=True)
            return carry
        lax.fori_loop(0, NWH, body, 0)
        pltpu.make_async_copy(rowsA, acc_sp.at[sidxA], ssA).wait()
        pltpu.make_async_copy(rowsB, acc_sp.at[sidxB], ssB).wait()
        pltpu.make_async_copy(albA, den_sp.at[sidxA], sdA).wait()
        pltpu.make_async_copy(albB, den_sp.at[sidxB], sdB).wait()
        plsc.subcore_barrier()
        _row_chunks(sid, lambda off, sz: pltpu.sync_copy(
            acc_sp.at[pl.ds(off, sz)], out_hbm.at[head, pl.ds(off, sz)]))

        def rden(off, sz):
            pltpu.sync_copy(den_sp.at[pl.ds(off, sz)],
                            bnc.at[pl.ds(0, sz)])
            pltpu.sync_copy(bnc.at[pl.ds(0, sz)],
                            deno_hbm.at[pl.ds(hoff + off, sz)])
        _row_chunks(sid, rden)
        plsc.subcore_barrier()


def _mm_body(x_ref, w_ref, o_ref):
    o_ref[...] = jnp.dot(x_ref[...], w_ref[...],
                         preferred_element_type=jnp.float32)


def _t1(x, W1):
    return pl.pallas_call(
        _mm_body,
        grid=(10,),
        in_specs=[pl.BlockSpec((N // 10, D), lambda i: (i, 0)),
                  pl.BlockSpec((D, D), lambda i: (0, 0))],
        out_specs=pl.BlockSpec((N // 10, D), lambda i: (i, 0)),
        out_shape=jax.ShapeDtypeStruct((N, D), jnp.float32),
    )(x, W1)


def _t3_body(aggA, aggB, b1, W2, al, ar, h0_o, elt_o, ert_o, m_o):
    h0 = jnp.maximum(aggA[...] + aggB[...] + b1[...], 0.0)
    h0_o[...] = h0
    w2 = W2[...]
    alv = al[...]
    arv = ar[...]
    cols_l = []
    cols_r = []
    for h in range(H):
        w2h = w2[:, h * O:(h + 1) * O]
        cols_l.append(lax.dot_general(w2h, alv[h:h + 1, :],
                                      (((1,), (1,)), ((), ()))))
        cols_r.append(lax.dot_general(w2h, arv[h:h + 1, :],
                                      (((1,), (1,)), ((), ()))))
    Wl = jnp.concatenate(cols_l, axis=1)
    Wr = jnp.concatenate(cols_r, axis=1)
    elt = lax.dot_general(Wl, h0, (((0,), (1,)), ((), ())))
    ert = lax.dot_general(Wr, h0, (((0,), (1,)), ((), ())))
    elt_o[...] = elt
    ert_o[...] = ert
    m = (jnp.max(elt, axis=1, keepdims=True)
         + jnp.max(ert, axis=1, keepdims=True))
    m_o[...] = jnp.broadcast_to(m, (H, 16))


def _t3(aggA, aggB, b1, W2, al, ar):
    return pl.pallas_call(
        _t3_body,
        out_shape=(jax.ShapeDtypeStruct((N, D), jnp.float32),
                   jax.ShapeDtypeStruct((H, N), jnp.float32),
                   jax.ShapeDtypeStruct((H, N), jnp.float32),
                   jax.ShapeDtypeStruct((H, 16), jnp.float32)),
    )(aggA, aggB, b1, W2, al, ar)


def _t5b_body(a_ref, o_ref):
    o_ref[...] = 1.0 / jnp.clip(a_ref[...], 1e-9, None)


def _t5b(den):
    return pl.pallas_call(
        _t5b_body,
        out_shape=jax.ShapeDtypeStruct((H, N), jnp.float32),
    )(den)


def _t7_body(acc_ref, ur_ref, W2_ref, b2_ref, o_ref):
    w2 = W2_ref[...]
    b2 = b2_ref[...]
    acc = acc_ref[...]
    ur = ur_ref[...]
    BN = acc.shape[1]
    s = jnp.zeros((BN, D), jnp.float32)
    for h in range(H):
        y = jnp.dot(acc[h], w2[:, h * O:(h + 1) * O],
                    preferred_element_type=jnp.float32)
        y = y * ur[:, h:h + 1] + b2[:, h * O:(h + 1) * O]
        s = s + jnp.maximum(y, 0.0)
    o_ref[...] = s * (1.0 / H)


def _t7(acc8, urec, W2, b2):
    BN = N // 10
    return pl.pallas_call(
        _t7_body,
        grid=(10,),
        in_specs=[pl.BlockSpec((H, BN, D), lambda i: (0, i, 0)),
                  pl.BlockSpec((BN, H), lambda i: (i, 0)),
                  pl.BlockSpec((D, H * O), lambda i: (0, 0)),
                  pl.BlockSpec((1, H * O), lambda i: (0, 0))],
        out_specs=pl.BlockSpec((BN, D), lambda i: (i, 0)),
        out_shape=jax.ShapeDtypeStruct((N, D), jnp.float32),
    )(acc8, urec, W2, b2)


def kernel(x, edge_index, edge_weight, W1, b1, W2, attn_l, attn_r, b2):
    src = edge_index[0]
    dst = edge_index[1]
    zeros_nd = jnp.zeros((N, D), jnp.float32)
    zeros_n = jnp.zeros((N,), jnp.float32)

    h = _t1(x, W1)
    aggA, aggB = _agg0_kernel(h, src, dst, edge_weight, zeros_nd)
    h0, elt, ert, mb = _t3(aggA, aggB, b1.reshape(1, D), W2,
                           attn_l, attn_r)
    m16 = mb.reshape(H * 16)
    elt_f = elt.reshape(H * N)
    ert_f = ert.reshape(H * N)
    acc8, den_f = _gat_kernel(h0, elt_f, ert_f, m16, src, dst,
                              zeros_nd, zeros_n)
    urec = _t5b(den_f.reshape(H, N))
    out = _t7(acc8, urec.T, W2, b2.reshape(1, H * O))
    return out


# speedup vs baseline: 22.7606x; 1.0142x over previous
import functools

import jax
import jax.numpy as jnp
from jax import lax
from jax.experimental import pallas as pl
from jax.experimental.pallas import tpu as pltpu
from jax.experimental.pallas import tpu_sc as plsc

N = 10000
E = 320000
D = 128
H = 8
O = 128

NC, NS = 2, 16
W = 80
ET2 = E // (NC * NS)
NW2 = ET2 // W
ET6 = E // NS
NW6 = ET6 // W
RCH = 624
HPC = H // NC
DQ = (HPC * N) // 8

_mesh = plsc.VectorSubcoreMesh(core_axis_name="c", subcore_axis_name="s")


def _row_chunks(sid, fn):
    off = pl.multiple_of(sid * RCH, 8)
    fn(off, RCH)

    @pl.when(sid == NS - 1)
    def _():
        fn(NS * RCH, N - NS * RCH)


NW2H = NW2 // 2


@functools.partial(
    pl.kernel,
    out_type=(jax.ShapeDtypeStruct((N, D), jnp.float32),
              jax.ShapeDtypeStruct((N, D), jnp.float32)),
    mesh=_mesh,
    scratch_types=[
        pltpu.VMEM_SHARED((N, D), jnp.float32),
        pltpu.VMEM((2, W), jnp.int32),
        pltpu.VMEM((2, W), jnp.int32),
        pltpu.VMEM((W,), jnp.int32),
        pltpu.VMEM((W,), jnp.int32),
        pltpu.VMEM((W,), jnp.float32),
        pltpu.VMEM((W,), jnp.float32),
        pltpu.VMEM((W, D), jnp.float32),
        pltpu.VMEM((W, D), jnp.float32),
        pltpu.SemaphoreType.DMA,
        pltpu.SemaphoreType.DMA,
        pltpu.SemaphoreType.DMA,
        pltpu.SemaphoreType.DMA,
        pltpu.SemaphoreType.DMA,
        pltpu.SemaphoreType.DMA,
    ],
)
def _agg0_kernel(h_hbm, src_hbm, dst_hbm, ew_hbm, zeros_hbm,
                 outa_hbm, outb_hbm, acc_sp,
                 idxA, idxB, sidxA, sidxB, ewA, ewB, rowsA, rowsB,
                 isemA, isemB, grA, grB, ssA, ssB):
    cid = lax.axis_index("c")
    sid = lax.axis_index("s")
    wid = cid * NS + sid
    _row_chunks(sid, lambda off, sz: pltpu.sync_copy(
        zeros_hbm.at[pl.ds(off, sz)], acc_sp.at[pl.ds(off, sz)]))
    plsc.subcore_barrier()

    def win_base(w):
        return pl.multiple_of(wid * ET2 + w * W, 8)

    def fetch(wi, buf, ewb, sem):
        b = win_base(wi)
        pltpu.async_copy(src_hbm.at[pl.ds(b, W)], buf.at[0], sem)
        pltpu.async_copy(dst_hbm.at[pl.ds(b, W)], buf.at[1], sem)
        pltpu.async_copy(ew_hbm.at[pl.ds(b, W)], ewb, sem)

    def wait_fetch(buf, ewb, sem):
        pltpu.make_async_copy(
            src_hbm.at[pl.ds(0, W)], buf.at[0], sem).wait()
        pltpu.make_async_copy(
            dst_hbm.at[pl.ds(0, W)], buf.at[1], sem).wait()
        pltpu.make_async_copy(ew_hbm.at[pl.ds(0, W)], ewb, sem).wait()

    def compute(idx2, ewb, rows, sidx):
        def scalef(k, c2):
            csl = pl.ds(k * 16, 16)
            sidx[csl] = idx2[1, csl]
            w16 = ewb[csl]
            for i in range(16):
                wgt = w16[i]
                row = k * 16 + i
                for j in range(D // 16):
                    sl = pl.ds(j * 16, 16)
                    rows[row, sl] = rows[row, sl] * wgt
            return c2
        lax.fori_loop(0, W // 16, scalef, 0)

    fetch(0, idxA, ewA, isemA)

    def body(w2, carry):
        e2 = w2 * 2
        wait_fetch(idxA, ewA, isemA)

        @pl.when(w2 > 0)
        def _():
            pltpu.make_async_copy(rowsA, acc_sp.at[sidxA], ssA).wait()
        ga = pltpu.async_copy(h_hbm.at[idxA.at[0]], rowsA, grA)
        fetch(e2 + 1, idxB, ewB, isemB)
        ga.wait()
        compute(idxA, ewA, rowsA, sidxA)
        pltpu.async_copy(rowsA, acc_sp.at[sidxA], ssA, add=True)
        wait_fetch(idxB, ewB, isemB)

        @pl.when(w2 > 0)
        def _():
            pltpu.make_async_copy(rowsB, acc_sp.at[sidxB], ssB).wait()
        gb = pltpu.async_copy(h_hbm.at[idxB.at[0]], rowsB, grB)
        fetch(e2 + 2, idxA, ewA, isemA)
        gb.wait()
        compute(idxB, ewB, rowsB, sidxB)
        pltpu.async_copy(rowsB, acc_sp.at[sidxB], ssB, add=True)
        return carry
    lax.fori_loop(0, NW2H, body, 0)
    wait_fetch(idxA, ewA, isemA)
    pltpu.make_async_copy(rowsA, acc_sp.at[sidxA], ssA).wait()
    ga = pltpu.async_copy(h_hbm.at[idxA.at[0]], rowsA, grA)
    ga.wait()
    compute(idxA, ewA, rowsA, sidxA)
    pltpu.async_copy(rowsA, acc_sp.at[sidxA], ssA, add=True)
    pltpu.make_async_copy(rowsA, acc_sp.at[sidxA], ssA).wait()
    pltpu.make_async_copy(rowsB, acc_sp.at[sidxB], ssB).wait()
    plsc.subcore_barrier()

    @pl.when(cid == 0)
    def _():
        _row_chunks(sid, lambda off, sz: pltpu.sync_copy(
            acc_sp.at[pl.ds(off, sz)], outa_hbm.at[pl.ds(off, sz)]))

    @pl.when(cid == 1)
    def _():
        _row_chunks(sid, lambda off, sz: pltpu.sync_copy(
            acc_sp.at[pl.ds(off, sz)], outb_hbm.at[pl.ds(off, sz)]))


NWH = NW6 // 2


@functools.partial(
    pl.kernel,
    out_type=(jax.ShapeDtypeStruct((H, N, D), jnp.float32),
              jax.ShapeDtypeStruct((H * N,), jnp.float32)),
    mesh=_mesh,
    scratch_types=[
        pltpu.VMEM_SHARED((N, D), jnp.float32),
        pltpu.VMEM_SHARED((N,), jnp.float32),
        pltpu.VMEM_SHARED((N,), jnp.float32),
        pltpu.VMEM_SHARED((N,), jnp.float32),
        pltpu.VMEM((2, W), jnp.int32),
        pltpu.VMEM((2, W), jnp.int32),
        pltpu.VMEM((W,), jnp.int32),
        pltpu.VMEM((W,), jnp.int32),
        pltpu.VMEM((W, D), jnp.float32),
        pltpu.VMEM((W, D), jnp.float32),
        pltpu.VMEM((W,), jnp.float32),
        pltpu.VMEM((W,), jnp.float32),
        pltpu.VMEM((W,), jnp.float32),
        pltpu.VMEM((W,), jnp.float32),
        pltpu.VMEM((W,), jnp.float32),
        pltpu.VMEM((W,), jnp.float32),
        pltpu.VMEM((16,), jnp.float32),
        pltpu.VMEM((RCH + 16,), jnp.float32),
        pltpu.SemaphoreType.DMA,
        pltpu.SemaphoreType.DMA,
        pltpu.SemaphoreType.DMA,
        pltpu.SemaphoreType.DMA,
        pltpu.SemaphoreType.DMA,
        pltpu.SemaphoreType.DMA,
        pltpu.SemaphoreType.DMA,
        pltpu.SemaphoreType.DMA,
        pltpu.SemaphoreType.DMA,
        pltpu.SemaphoreType.DMA,
        pltpu.SemaphoreType.DMA,
        pltpu.SemaphoreType.DMA,
    ],
)
def _gat_kernel(h0_hbm, elt_hbm, ert_hbm, m_hbm, src_hbm, dst_hbm,
                zeros_hbm, zerosn_hbm, out_hbm, deno_hbm,
                acc_sp, el1_sp, er1_sp, den_sp,
                idxA, idxB, sidxA, sidxB, rowsA, rowsB,
                elbA, elbB, erbA, erbB, albA, albB, mv, bnc,
                isemA, isemB, grA, grB, geA, geB, gfA, gfB, ssA, ssB,
                sdA, sdB):
    cid = lax.axis_index("c")
    sid = lax.axis_index("s")

    def win_base(w):
        return pl.multiple_of(sid * ET6 + w * W, 8)

    def alpha_part(idx2, elb, erb, sidx, alb, mhead):
        def chunkf(k, c2):
            csl = pl.ds(k * 16, 16)
            z = elb[csl] + erb[csl]
            e = jnp.maximum(z, z * 0.2)
            alb[csl] = jnp.exp(e - mhead)
            sidx[csl] = idx2[1, csl]
            return c2
        lax.fori_loop(0, W // 16, chunkf, 0)

    def scale_part(rows, alb):
        def scalef(k, c2):
            a16 = alb[pl.ds(k * 16, 16)]
            for i in range(16):
                a = a16[i]
                row = k * 16 + i
                for j in range(D // 16):
                    sl = pl.ds(j * 16, 16)
                    rows[row, sl] = rows[row, sl] * a
            return c2
        lax.fori_loop(0, W // 16, scalef, 0)

    for p in range(HPC):
        head = cid * HPC + p
        hoff = pl.multiple_of(head * N, 8)
        pltpu.sync_copy(m_hbm.at[pl.ds(pl.multiple_of(head * 16, 8), 16)],
                        mv)
        _row_chunks(sid, lambda off, sz: pltpu.sync_copy(
            zeros_hbm.at[pl.ds(off, sz)], acc_sp.at[pl.ds(off, sz)]))

        def zden(off, sz):
            pltpu.sync_copy(zerosn_hbm.at[pl.ds(off, sz)],
                            bnc.at[pl.ds(0, sz)])
            pltpu.sync_copy(bnc.at[pl.ds(0, sz)],
                            den_sp.at[pl.ds(off, sz)])
        _row_chunks(sid, zden)

        def stage(tbl_hbm, dst_sp):
            def cp(off, sz):
                pltpu.sync_copy(tbl_hbm.at[pl.ds(hoff + off, sz)],
                                bnc.at[pl.ds(0, sz)])
                pltpu.sync_copy(bnc.at[pl.ds(0, sz)],
                                dst_sp.at[pl.ds(off, sz)])
            _row_chunks(sid, cp)
        stage(elt_hbm, el1_sp)
        stage(ert_hbm, er1_sp)
        plsc.subcore_barrier()
        mhead = mv[:]

        def fetch_idx(wi, buf, sem):
            b = win_base(wi)
            pltpu.async_copy(src_hbm.at[pl.ds(b, W)], buf.at[0], sem)
            pltpu.async_copy(dst_hbm.at[pl.ds(b, W)], buf.at[1], sem)

        def wait_idx(buf, sem):
            pltpu.make_async_copy(
                src_hbm.at[pl.ds(0, W)], buf.at[0], sem).wait()
            pltpu.make_async_copy(
                dst_hbm.at[pl.ds(0, W)], buf.at[1], sem).wait()

        fetch_idx(0, idxA, isemA)

        def body(w2, carry):
            e2 = w2 * 2
            wait_idx(idxA, isemA)

            @pl.when(w2 > 0)
            def _():
                pltpu.make_async_copy(
                    rowsA, acc_sp.at[sidxA], ssA).wait()
                pltpu.make_async_copy(
                    albA, den_sp.at[sidxA], sdA).wait()
            ga = pltpu.async_copy(h0_hbm.at[idxA.at[0]], rowsA, grA)
            gb = pltpu.async_copy(el1_sp.at[idxA.at[0]], elbA, geA)
            gc = pltpu.async_copy(er1_sp.at[idxA.at[1]], erbA, gfA)
            fetch_idx(e2 + 1, idxB, isemB)
            gb.wait()
            gc.wait()
            alpha_part(idxA, elbA, erbA, sidxA, albA, mhead)
            ga.wait()
            scale_part(rowsA, albA)
            pltpu.async_copy(rowsA, acc_sp.at[sidxA], ssA, add=True)
            pltpu.async_copy(albA, den_sp.at[sidxA], sdA, add=True)
            wait_idx(idxB, isemB)

            @pl.when(w2 > 0)
            def _():
                pltpu.make_async_copy(
                    rowsB, acc_sp.at[sidxB], ssB).wait()
                pltpu.make_async_copy(
                    albB, den_sp.at[sidxB], sdB).wait()
            ga2 = pltpu.async_copy(h0_hbm.at[idxB.at[0]], rowsB, grB)
            gb2 = pltpu.async_copy(el1_sp.at[idxB.at[0]], elbB, geB)
            gc2 = pltpu.async_copy(er1_sp.at[idxB.at[1]], erbB, gfB)

            @pl.when(w2 < NWH - 1)
            def _():
                fetch_idx(e2 + 2, idxA, isemA)
            gb2.wait()
            gc2.wait()
            alpha_part(idxB, elbB, erbB, sidxB, albB, mhead)
            ga2.wait()
            scale_part(rowsB, albB)
            pltpu.async_copy(rowsB, acc_sp.at[sidxB], ssB, add=True)
            pltpu.async_copy(albB, den_sp.at[sidxB], sdB, add=True)
            return carry
        lax.fori_loop(0, NWH, body, 0)
        pltpu.make_async_copy(rowsA, acc_sp.at[sidxA], ssA).wait()
        pltpu.make_async_copy(rowsB, acc_sp.at[sidxB], ssB).wait()
        pltpu.make_async_copy(albA, den_sp.at[sidxA], sdA).wait()
        pltpu.make_async_copy(albB, den_sp.at[sidxB], sdB).wait()
        plsc.subcore_barrier()
        _row_chunks(sid, lambda off, sz: pltpu.sync_copy(
            acc_sp.at[pl.ds(off, sz)], out_hbm.at[head, pl.ds(off, sz)]))

        def rden(off, sz):
            pltpu.sync_copy(den_sp.at[pl.ds(off, sz)],
                            bnc.at[pl.ds(0, sz)])
            pltpu.sync_copy(bnc.at[pl.ds(0, sz)],
                            deno_hbm.at[pl.ds(hoff + off, sz)])
        _row_chunks(sid, rden)
        plsc.subcore_barrier()


def _mm_body(x_ref, w_ref, o_ref):
    o_ref[...] = jnp.dot(x_ref[...], w_ref[...],
                         preferred_element_type=jnp.float32)


def _t1(x, W1):
    return pl.pallas_call(
        _mm_body,
        grid=(10,),
        in_specs=[pl.BlockSpec((N // 10, D), lambda i: (i, 0)),
                  pl.BlockSpec((D, D), lambda i: (0, 0))],
        out_specs=pl.BlockSpec((N // 10, D), lambda i: (i, 0)),
        out_shape=jax.ShapeDtypeStruct((N, D), jnp.float32),
    )(x, W1)


def _t3_body(aggA, aggB, b1, W2, al, ar, h0_o, elt_o, ert_o, m_o):
    h0 = jnp.maximum(aggA[...] + aggB[...] + b1[...], 0.0)
    h0_o[...] = h0
    w2 = W2[...]
    alv = al[...]
    arv = ar[...]
    cols_l = []
    cols_r = []
    for h in range(H):
        w2h = w2[:, h * O:(h + 1) * O]
        cols_l.append(lax.dot_general(w2h, alv[h:h + 1, :],
                                      (((1,), (1,)), ((), ()))))
        cols_r.append(lax.dot_general(w2h, arv[h:h + 1, :],
                                      (((1,), (1,)), ((), ()))))
    Wl = jnp.concatenate(cols_l, axis=1)
    Wr = jnp.concatenate(cols_r, axis=1)
    elt = lax.dot_general(Wl, h0, (((0,), (1,)), ((), ())))
    ert = lax.dot_general(Wr, h0, (((0,), (1,)), ((), ())))
    elt_o[...] = elt
    ert_o[...] = ert
    m = (jnp.max(elt, axis=1, keepdims=True)
         + jnp.max(ert, axis=1, keepdims=True))
    m_o[...] = jnp.broadcast_to(m, (H, 16))


def _t3(aggA, aggB, b1, W2, al, ar):
    return pl.pallas_call(
        _t3_body,
        out_shape=(jax.ShapeDtypeStruct((N, D), jnp.float32),
                   jax.ShapeDtypeStruct((H, N), jnp.float32),
                   jax.ShapeDtypeStruct((H, N), jnp.float32),
                   jax.ShapeDtypeStruct((H, 16), jnp.float32)),
    )(aggA, aggB, b1, W2, al, ar)


def _t7_body(acc_ref, ur_ref, W2_ref, b2_ref, o_ref):
    w2 = W2_ref[...]
    b2 = b2_ref[...]
    acc = acc_ref[...]
    ur = 1.0 / jnp.clip(ur_ref[...], 1e-9, None)
    BN = acc.shape[1]
    s = jnp.zeros((BN, D), jnp.float32)
    for h in range(H):
        y = jnp.dot(acc[h], w2[:, h * O:(h + 1) * O],
                    preferred_element_type=jnp.float32)
        y = y * ur[:, h:h + 1] + b2[:, h * O:(h + 1) * O]
        s = s + jnp.maximum(y, 0.0)
    o_ref[...] = s * (1.0 / H)


def _t7(acc8, urec, W2, b2):
    BN = N // 10
    return pl.pallas_call(
        _t7_body,
        grid=(10,),
        in_specs=[pl.BlockSpec((H, BN, D), lambda i: (0, i, 0)),
                  pl.BlockSpec((BN, H), lambda i: (i, 0)),
                  pl.BlockSpec((D, H * O), lambda i: (0, 0)),
                  pl.BlockSpec((1, H * O), lambda i: (0, 0))],
        out_specs=pl.BlockSpec((BN, D), lambda i: (i, 0)),
        out_shape=jax.ShapeDtypeStruct((N, D), jnp.float32),
    )(acc8, urec, W2, b2)


def kernel(x, edge_index, edge_weight, W1, b1, W2, attn_l, attn_r, b2):
    src = edge_index[0]
    dst = edge_index[1]
    zeros_nd = jnp.zeros((N, D), jnp.float32)
    zeros_n = jnp.zeros((N,), jnp.float32)

    h = _t1(x, W1)
    aggA, aggB = _agg0_kernel(h, src, dst, edge_weight, zeros_nd)
    h0, elt, ert, mb = _t3(aggA, aggB, b1.reshape(1, D), W2,
                           attn_l, attn_r)
    m16 = mb.reshape(H * 16)
    elt_f = elt.reshape(H * N)
    ert_f = ert.reshape(H * N)
    acc8, den_f = _gat_kernel(h0, elt_f, ert_f, m16, src, dst,
                              zeros_nd, zeros_n)
    out = _t7(acc8, den_f.reshape(H, N).T, W2, b2.reshape(1, H * O))
    return out


# speedup vs baseline: 24.7240x; 1.0863x over previous
import functools

import jax
import jax.numpy as jnp
from jax import lax
from jax.experimental import pallas as pl
from jax.experimental.pallas import tpu as pltpu
from jax.experimental.pallas import tpu_sc as plsc

N = 10000
E = 320000
D = 128
H = 8
O = 128

NC, NS = 2, 16
W = 80
ET2 = E // (NC * NS)
NW2 = ET2 // W
ET6 = E // NS
NW6 = ET6 // W
RCH = 624
HPC = H // NC

_mesh = plsc.VectorSubcoreMesh(core_axis_name="c", subcore_axis_name="s")


def _row_chunks(sid, fn):
    off = pl.multiple_of(sid * RCH, 8)
    fn(off, RCH)

    @pl.when(sid == NS - 1)
    def _():
        fn(NS * RCH, N - NS * RCH)


NW2H = NW2 // 2


@functools.partial(
    pl.kernel,
    out_type=(jax.ShapeDtypeStruct((N, D), jnp.float32),
              jax.ShapeDtypeStruct((N, D), jnp.float32)),
    mesh=_mesh,
    scratch_types=[
        pltpu.VMEM_SHARED((N, D), jnp.float32),
        pltpu.VMEM((2, W), jnp.int32),
        pltpu.VMEM((2, W), jnp.int32),
        pltpu.VMEM((W,), jnp.int32),
        pltpu.VMEM((W,), jnp.int32),
        pltpu.VMEM((W,), jnp.float32),
        pltpu.VMEM((W,), jnp.float32),
        pltpu.VMEM((W, D), jnp.float32),
        pltpu.VMEM((W, D), jnp.float32),
        pltpu.SemaphoreType.DMA,
        pltpu.SemaphoreType.DMA,
        pltpu.SemaphoreType.DMA,
        pltpu.SemaphoreType.DMA,
        pltpu.SemaphoreType.DMA,
        pltpu.SemaphoreType.DMA,
    ],
)
def _agg0_kernel(h_hbm, src_hbm, dst_hbm, ew_hbm, zeros_hbm,
                 outa_hbm, outb_hbm, acc_sp,
                 idxA, idxB, sidxA, sidxB, ewA, ewB, rowsA, rowsB,
                 isemA, isemB, grA, grB, ssA, ssB):
    cid = lax.axis_index("c")
    sid = lax.axis_index("s")
    wid = cid * NS + sid
    _row_chunks(sid, lambda off, sz: pltpu.sync_copy(
        zeros_hbm.at[pl.ds(off, sz)], acc_sp.at[pl.ds(off, sz)]))
    plsc.subcore_barrier()

    def win_base(w):
        return pl.multiple_of(wid * ET2 + w * W, 8)

    def fetch(wi, buf, ewb, sem):
        b = win_base(wi)
        pltpu.async_copy(src_hbm.at[pl.ds(b, W)], buf.at[0], sem)
        pltpu.async_copy(dst_hbm.at[pl.ds(b, W)], buf.at[1], sem)
        pltpu.async_copy(ew_hbm.at[pl.ds(b, W)], ewb, sem)

    def wait_fetch(buf, ewb, sem):
        pltpu.make_async_copy(
            src_hbm.at[pl.ds(0, W)], buf.at[0], sem).wait()
        pltpu.make_async_copy(
            dst_hbm.at[pl.ds(0, W)], buf.at[1], sem).wait()
        pltpu.make_async_copy(ew_hbm.at[pl.ds(0, W)], ewb, sem).wait()

    def compute(idx2, ewb, rows, sidx):
        def scalef(k, c2):
            csl = pl.ds(k * 16, 16)
            sidx[csl] = idx2[1, csl]
            w16 = ewb[csl]
            for i in range(16):
                wgt = w16[i]
                row = k * 16 + i
                for j in range(D // 16):
                    sl = pl.ds(j * 16, 16)
                    rows[row, sl] = rows[row, sl] * wgt
            return c2
        lax.fori_loop(0, W // 16, scalef, 0)

    fetch(0, idxA, ewA, isemA)

    def body(w2, carry):
        e2 = w2 * 2
        wait_fetch(idxA, ewA, isemA)

        @pl.when(w2 > 0)
        def _():
            pltpu.make_async_copy(rowsA, acc_sp.at[sidxA], ssA).wait()
        ga = pltpu.async_copy(h_hbm.at[idxA.at[0]], rowsA, grA)
        fetch(e2 + 1, idxB, ewB, isemB)
        ga.wait()
        compute(idxA, ewA, rowsA, sidxA)
        pltpu.async_copy(rowsA, acc_sp.at[sidxA], ssA, add=True)
        wait_fetch(idxB, ewB, isemB)

        @pl.when(w2 > 0)
        def _():
            pltpu.make_async_copy(rowsB, acc_sp.at[sidxB], ssB).wait()
        gb = pltpu.async_copy(h_hbm.at[idxB.at[0]], rowsB, grB)
        fetch(e2 + 2, idxA, ewA, isemA)
        gb.wait()
        compute(idxB, ewB, rowsB, sidxB)
        pltpu.async_copy(rowsB, acc_sp.at[sidxB], ssB, add=True)
        return carry
    lax.fori_loop(0, NW2H, body, 0)
    wait_fetch(idxA, ewA, isemA)
    pltpu.make_async_copy(rowsA, acc_sp.at[sidxA], ssA).wait()
    ga = pltpu.async_copy(h_hbm.at[idxA.at[0]], rowsA, grA)
    ga.wait()
    compute(idxA, ewA, rowsA, sidxA)
    pltpu.async_copy(rowsA, acc_sp.at[sidxA], ssA, add=True)
    pltpu.make_async_copy(rowsA, acc_sp.at[sidxA], ssA).wait()
    pltpu.make_async_copy(rowsB, acc_sp.at[sidxB], ssB).wait()
    plsc.subcore_barrier()

    @pl.when(cid == 0)
    def _():
        _row_chunks(sid, lambda off, sz: pltpu.sync_copy(
            acc_sp.at[pl.ds(off, sz)], outa_hbm.at[pl.ds(off, sz)]))

    @pl.when(cid == 1)
    def _():
        _row_chunks(sid, lambda off, sz: pltpu.sync_copy(
            acc_sp.at[pl.ds(off, sz)], outb_hbm.at[pl.ds(off, sz)]))


NWH = NW6 // 2


@functools.partial(
    pl.kernel,
    out_type=(jax.ShapeDtypeStruct((H, N, D), jnp.float32),
              jax.ShapeDtypeStruct((H * N,), jnp.float32)),
    mesh=_mesh,
    scratch_types=[
        pltpu.VMEM_SHARED((N, D), jnp.float32),
        pltpu.VMEM_SHARED((N,), jnp.float32),
        pltpu.VMEM_SHARED((N,), jnp.float32),
        pltpu.VMEM_SHARED((N,), jnp.float32),
        pltpu.VMEM((2, W), jnp.int32),
        pltpu.VMEM((2, W), jnp.int32),
        pltpu.VMEM((W,), jnp.int32),
        pltpu.VMEM((W,), jnp.int32),
        pltpu.VMEM((W, D), jnp.float32),
        pltpu.VMEM((W, D), jnp.float32),
        pltpu.VMEM((W,), jnp.float32),
        pltpu.VMEM((W,), jnp.float32),
        pltpu.VMEM((W,), jnp.float32),
        pltpu.VMEM((W,), jnp.float32),
        pltpu.VMEM((W,), jnp.float32),
        pltpu.VMEM((W,), jnp.float32),
        pltpu.VMEM((16,), jnp.float32),
        pltpu.VMEM((RCH + 16,), jnp.float32),
        pltpu.SemaphoreType.DMA,
        pltpu.SemaphoreType.DMA,
        pltpu.SemaphoreType.DMA,
        pltpu.SemaphoreType.DMA,
        pltpu.SemaphoreType.DMA,
        pltpu.SemaphoreType.DMA,
        pltpu.SemaphoreType.DMA,
        pltpu.SemaphoreType.DMA,
        pltpu.SemaphoreType.DMA,
        pltpu.SemaphoreType.DMA,
        pltpu.SemaphoreType.DMA,
        pltpu.SemaphoreType.DMA,
    ],
)
def _gat_kernel(h0_hbm, elt_hbm, ert_hbm, m_hbm, src_hbm, dst_hbm,
                zeros_hbm, zerosn_hbm, out_hbm, deno_hbm,
                acc_sp, el1_sp, er1_sp, den_sp,
                idxA, idxB, sidxA, sidxB, rowsA, rowsB,
                elbA, elbB, erbA, erbB, albA, albB, mv, bnc,
                isemA, isemB, grA, grB, geA, geB, gfA, gfB, ssA, ssB,
                sdA, sdB):
    cid = lax.axis_index("c")
    sid = lax.axis_index("s")

    def win_base(w):
        return pl.multiple_of(sid * ET6 + w * W, 8)

    def alpha_part(idx2, elb, erb, sidx, alb, mhead):
        def chunkf(k, c2):
            csl = pl.ds(k * 16, 16)
            z = elb[csl] + erb[csl]
            e = jnp.maximum(z, z * 0.2)
            alb[csl] = jnp.exp(e - mhead)
            sidx[csl] = idx2[1, csl]
            return c2
        lax.fori_loop(0, W // 16, chunkf, 0)

    def scale_part(rows, alb, k0, k1):
        def scalef(k, c2):
            a16 = alb[pl.ds(k * 16, 16)]
            for i in range(16):
                a = a16[i]
                row = k * 16 + i
                for j in range(D // 16):
                    sl = pl.ds(j * 16, 16)
                    rows[row, sl] = rows[row, sl] * a
            return c2
        lax.fori_loop(k0, k1, scalef, 0)

    for p in range(HPC):
        head = cid * HPC + p
        hoff = pl.multiple_of(head * N, 8)
        pltpu.sync_copy(m_hbm.at[pl.ds(pl.multiple_of(head * 16, 8), 16)],
                        mv)
        _row_chunks(sid, lambda off, sz: pltpu.sync_copy(
            zeros_hbm.at[pl.ds(off, sz)], acc_sp.at[pl.ds(off, sz)]))

        def zden(off, sz):
            pltpu.sync_copy(zerosn_hbm.at[pl.ds(off, sz)],
                            bnc.at[pl.ds(0, sz)])
            pltpu.sync_copy(bnc.at[pl.ds(0, sz)],
                            den_sp.at[pl.ds(off, sz)])
        _row_chunks(sid, zden)

        def stage(tbl_hbm, dst_sp):
            def cp(off, sz):
                pltpu.sync_copy(tbl_hbm.at[pl.ds(hoff + off, sz)],
                                bnc.at[pl.ds(0, sz)])
                pltpu.sync_copy(bnc.at[pl.ds(0, sz)],
                                dst_sp.at[pl.ds(off, sz)])
            _row_chunks(sid, cp)
        stage(elt_hbm, el1_sp)
        stage(ert_hbm, er1_sp)
        plsc.subcore_barrier()
        mhead = mv[:]

        def fetch_idx(wi, buf, sem):
            b = win_base(wi)
            pltpu.async_copy(src_hbm.at[pl.ds(b, W)], buf.at[0], sem)
            pltpu.async_copy(dst_hbm.at[pl.ds(b, W)], buf.at[1], sem)

        def wait_idx(buf, sem):
            pltpu.make_async_copy(
                src_hbm.at[pl.ds(0, W)], buf.at[0], sem).wait()
            pltpu.make_async_copy(
                dst_hbm.at[pl.ds(0, W)], buf.at[1], sem).wait()

        fetch_idx(0, idxA, isemA)

        def body(w2, carry):
            e2 = w2 * 2
            wait_idx(idxA, isemA)

            @pl.when(w2 > 0)
            def _():
                pltpu.make_async_copy(
                    rowsA, acc_sp.at[sidxA], ssA).wait()
                pltpu.make_async_copy(
                    albA, den_sp.at[sidxA], sdA).wait()
            ga1 = pltpu.async_copy(h0_hbm.at[idxA.at[0, pl.ds(0, 48)]],
                                   rowsA.at[pl.ds(0, 48)], grA)
            ga2 = pltpu.async_copy(h0_hbm.at[idxA.at[0, pl.ds(48, 32)]],
                                   rowsA.at[pl.ds(48, 32)], grA)
            gb = pltpu.async_copy(el1_sp.at[idxA.at[0]], elbA, geA)
            gc = pltpu.async_copy(er1_sp.at[idxA.at[1]], erbA, gfA)
            fetch_idx(e2 + 1, idxB, isemB)
            gb.wait()
            gc.wait()
            alpha_part(idxA, elbA, erbA, sidxA, albA, mhead)
            ga1.wait()
            scale_part(rowsA, albA, 0, 3)
            ga2.wait()
            scale_part(rowsA, albA, 3, W // 16)
            pltpu.async_copy(rowsA, acc_sp.at[sidxA], ssA, add=True)
            pltpu.async_copy(albA, den_sp.at[sidxA], sdA, add=True)
            wait_idx(idxB, isemB)

            @pl.when(w2 > 0)
            def _():
                pltpu.make_async_copy(
                    rowsB, acc_sp.at[sidxB], ssB).wait()
                pltpu.make_async_copy(
                    albB, den_sp.at[sidxB], sdB).wait()
            gd1 = pltpu.async_copy(h0_hbm.at[idxB.at[0, pl.ds(0, 48)]],
                                   rowsB.at[pl.ds(0, 48)], grB)
            gd2 = pltpu.async_copy(h0_hbm.at[idxB.at[0, pl.ds(48, 32)]],
                                   rowsB.at[pl.ds(48, 32)], grB)
            gb2 = pltpu.async_copy(el1_sp.at[idxB.at[0]], elbB, geB)
            gc2 = pltpu.async_copy(er1_sp.at[idxB.at[1]], erbB, gfB)

            @pl.when(w2 < NWH - 1)
            def _():
                fetch_idx(e2 + 2, idxA, isemA)
            gb2.wait()
            gc2.wait()
            alpha_part(idxB, elbB, erbB, sidxB, albB, mhead)
            gd1.wait()
            scale_part(rowsB, albB, 0, 3)
            gd2.wait()
            scale_part(rowsB, albB, 3, W // 16)
            pltpu.async_copy(rowsB, acc_sp.at[sidxB], ssB, add=True)
            pltpu.async_copy(albB, den_sp.at[sidxB], sdB, add=True)
            return carry
        lax.fori_loop(0, NWH, body, 0)
        pltpu.make_async_copy(rowsA, acc_sp.at[sidxA], ssA).wait()
        pltpu.make_async_copy(rowsB, acc_sp.at[sidxB], ssB).wait()
        pltpu.make_async_copy(albA, den_sp.at[sidxA], sdA).wait()
        pltpu.make_async_copy(albB, den_sp.at[sidxB], sdB).wait()
        plsc.subcore_barrier()
        _row_chunks(sid, lambda off, sz: pltpu.sync_copy(
            acc_sp.at[pl.ds(off, sz)], out_hbm.at[head, pl.ds(off, sz)]))

        def rden(off, sz):
            pltpu.sync_copy(den_sp.at[pl.ds(off, sz)],
                            bnc.at[pl.ds(0, sz)])
            pltpu.sync_copy(bnc.at[pl.ds(0, sz)],
                            deno_hbm.at[pl.ds(hoff + off, sz)])
        _row_chunks(sid, rden)
        plsc.subcore_barrier()


def _mm_body(x_ref, w_ref, o_ref):
    o_ref[...] = jnp.dot(x_ref[...], w_ref[...],
                         preferred_element_type=jnp.float32)


def _t1(x, W1):
    return pl.pallas_call(
        _mm_body,
        grid=(10,),
        in_specs=[pl.BlockSpec((N // 10, D), lambda i: (i, 0)),
                  pl.BlockSpec((D, D), lambda i: (0, 0))],
        out_specs=pl.BlockSpec((N // 10, D), lambda i: (i, 0)),
        out_shape=jax.ShapeDtypeStruct((N, D), jnp.float32),
    )(x, W1)


def _t3_body(aggA, aggB, b1, W2, al, ar, h0_o, elt_o, ert_o, m_o):
    h0 = jnp.maximum(aggA[...] + aggB[...] + b1[...], 0.0)
    h0_o[...] = h0
    w2 = W2[...]
    alv = al[...]
    arv = ar[...]
    cols_l = []
    cols_r = []
    for h in range(H):
        w2h = w2[:, h * O:(h + 1) * O]
        cols_l.append(lax.dot_general(w2h, alv[h:h + 1, :],
                                      (((1,), (1,)), ((), ()))))
        cols_r.append(lax.dot_general(w2h, arv[h:h + 1, :],
                                      (((1,), (1,)), ((), ()))))
    Wl = jnp.concatenate(cols_l, axis=1)
    Wr = jnp.concatenate(cols_r, axis=1)
    elt = lax.dot_general(Wl, h0, (((0,), (1,)), ((), ())))
    ert = lax.dot_general(Wr, h0, (((0,), (1,)), ((), ())))
    elt_o[...] = elt
    ert_o[...] = ert
    m = (jnp.max(elt, axis=1, keepdims=True)
         + jnp.max(ert, axis=1, keepdims=True))
    m_o[...] = jnp.broadcast_to(m, (H, 16))


def _t3(aggA, aggB, b1, W2, al, ar):
    return pl.pallas_call(
        _t3_body,
        out_shape=(jax.ShapeDtypeStruct((N, D), jnp.float32),
                   jax.ShapeDtypeStruct((H, N), jnp.float32),
                   jax.ShapeDtypeStruct((H, N), jnp.float32),
                   jax.ShapeDtypeStruct((H, 16), jnp.float32)),
    )(aggA, aggB, b1, W2, al, ar)


def _t7_body(acc_ref, ur_ref, W2_ref, b2_ref, o_ref):
    w2 = W2_ref[...]
    b2 = b2_ref[...]
    acc = acc_ref[...]
    ur = 1.0 / jnp.clip(ur_ref[...], 1e-9, None)
    BN = acc.shape[1]
    s = jnp.zeros((BN, D), jnp.float32)
    for h in range(H):
        y = jnp.dot(acc[h], w2[:, h * O:(h + 1) * O],
                    preferred_element_type=jnp.float32)
        y = y * ur[:, h:h + 1] + b2[:, h * O:(h + 1) * O]
        s = s + jnp.maximum(y, 0.0)
    o_ref[...] = s * (1.0 / H)


def _t7(acc8, urec, W2, b2):
    BN = N // 10
    return pl.pallas_call(
        _t7_body,
        grid=(10,),
        in_specs=[pl.BlockSpec((H, BN, D), lambda i: (0, i, 0)),
                  pl.BlockSpec((BN, H), lambda i: (i, 0)),
                  pl.BlockSpec((D, H * O), lambda i: (0, 0)),
                  pl.BlockSpec((1, H * O), lambda i: (0, 0))],
        out_specs=pl.BlockSpec((BN, D), lambda i: (i, 0)),
        out_shape=jax.ShapeDtypeStruct((N, D), jnp.float32),
    )(acc8, urec, W2, b2)


def kernel(x, edge_index, edge_weight, W1, b1, W2, attn_l, attn_r, b2):
    src = edge_index[0]
    dst = edge_index[1]
    zeros_nd = jnp.zeros((N, D), jnp.float32)
    zeros_n = jnp.zeros((N,), jnp.float32)

    h = _t1(x, W1)
    aggA, aggB = _agg0_kernel(h, src, dst, edge_weight, zeros_nd)
    h0, elt, ert, mb = _t3(aggA, aggB, b1.reshape(1, D), W2,
                           attn_l, attn_r)
    m16 = mb.reshape(H * 16)
    elt_f = elt.reshape(H * N)
    ert_f = ert.reshape(H * N)
    acc8, den_f = _gat_kernel(h0, elt_f, ert_f, m16, src, dst,
                              zeros_nd, zeros_n)
    out = _t7(acc8, den_f.reshape(H, N).T, W2, b2.reshape(1, H * O))
    return out


# speedup vs baseline: 25.0289x; 1.0123x over previous
import functools

import jax
import jax.numpy as jnp
from jax import lax
from jax.experimental import pallas as pl
from jax.experimental.pallas import tpu as pltpu
from jax.experimental.pallas import tpu_sc as plsc

N = 10000
E = 320000
D = 128
H = 8
O = 128

NC, NS = 2, 16
W = 80
ET2 = E // (NC * NS)
NW2 = ET2 // W
ET6 = E // NS
NW6 = ET6 // W
RCH = 624
HPC = H // NC

_mesh = plsc.VectorSubcoreMesh(core_axis_name="c", subcore_axis_name="s")


def _row_chunks(sid, fn):
    off = pl.multiple_of(sid * RCH, 8)
    fn(off, RCH)

    @pl.when(sid == NS - 1)
    def _():
        fn(NS * RCH, N - NS * RCH)


NW2H = NW2 // 2


@functools.partial(
    pl.kernel,
    out_type=(jax.ShapeDtypeStruct((N, D), jnp.float32),
              jax.ShapeDtypeStruct((N, D), jnp.float32)),
    mesh=_mesh,
    scratch_types=[
        pltpu.VMEM_SHARED((N, D), jnp.float32),
        pltpu.VMEM((2, W), jnp.int32),
        pltpu.VMEM((2, W), jnp.int32),
        pltpu.VMEM((W,), jnp.int32),
        pltpu.VMEM((W,), jnp.int32),
        pltpu.VMEM((W,), jnp.float32),
        pltpu.VMEM((W,), jnp.float32),
        pltpu.VMEM((W, D), jnp.float32),
        pltpu.VMEM((W, D), jnp.float32),
        pltpu.SemaphoreType.DMA,
        pltpu.SemaphoreType.DMA,
        pltpu.SemaphoreType.DMA,
        pltpu.SemaphoreType.DMA,
        pltpu.SemaphoreType.DMA,
        pltpu.SemaphoreType.DMA,
    ],
)
def _agg0_kernel(h_hbm, src_hbm, dst_hbm, ew_hbm, zeros_hbm,
                 outa_hbm, outb_hbm, acc_sp,
                 idxA, idxB, sidxA, sidxB, ewA, ewB, rowsA, rowsB,
                 isemA, isemB, grA, grB, ssA, ssB):
    cid = lax.axis_index("c")
    sid = lax.axis_index("s")
    wid = cid * NS + sid
    _row_chunks(sid, lambda off, sz: pltpu.sync_copy(
        zeros_hbm.at[pl.ds(off, sz)], acc_sp.at[pl.ds(off, sz)]))
    plsc.subcore_barrier()

    def win_base(w):
        return pl.multiple_of(wid * ET2 + w * W, 8)

    def fetch(wi, buf, ewb, sem):
        b = win_base(wi)
        pltpu.async_copy(src_hbm.at[pl.ds(b, W)], buf.at[0], sem)
        pltpu.async_copy(dst_hbm.at[pl.ds(b, W)], buf.at[1], sem)
        pltpu.async_copy(ew_hbm.at[pl.ds(b, W)], ewb, sem)

    def wait_fetch(buf, ewb, sem):
        pltpu.make_async_copy(
            src_hbm.at[pl.ds(0, W)], buf.at[0], sem).wait()
        pltpu.make_async_copy(
            dst_hbm.at[pl.ds(0, W)], buf.at[1], sem).wait()
        pltpu.make_async_copy(ew_hbm.at[pl.ds(0, W)], ewb, sem).wait()

    def compute(idx2, ewb, rows, sidx, k0, k1):
        def scalef(k, c2):
            csl = pl.ds(k * 16, 16)
            sidx[csl] = idx2[1, csl]
            w16 = ewb[csl]
            for i in range(16):
                wgt = w16[i]
                row = k * 16 + i
                for j in range(D // 16):
                    sl = pl.ds(j * 16, 16)
                    rows[row, sl] = rows[row, sl] * wgt
            return c2
        lax.fori_loop(k0, k1, scalef, 0)

    def gather_rows(idx2, rows, sem):
        g1 = pltpu.async_copy(h_hbm.at[idx2.at[0, pl.ds(0, 48)]],
                              rows.at[pl.ds(0, 48)], sem)
        g2 = pltpu.async_copy(h_hbm.at[idx2.at[0, pl.ds(48, 32)]],
                              rows.at[pl.ds(48, 32)], sem)
        return g1, g2

    def run_window(idx2, ewb, rows, sidx, g1, g2):
        g1.wait()
        compute(idx2, ewb, rows, sidx, 0, 3)
        g2.wait()
        compute(idx2, ewb, rows, sidx, 3, W // 16)

    fetch(0, idxA, ewA, isemA)

    def body(w2, carry):
        e2 = w2 * 2
        wait_fetch(idxA, ewA, isemA)

        @pl.when(w2 > 0)
        def _():
            pltpu.make_async_copy(rowsA, acc_sp.at[sidxA], ssA).wait()
        ga1, ga2 = gather_rows(idxA, rowsA, grA)
        fetch(e2 + 1, idxB, ewB, isemB)
        run_window(idxA, ewA, rowsA, sidxA, ga1, ga2)
        pltpu.async_copy(rowsA, acc_sp.at[sidxA], ssA, add=True)
        wait_fetch(idxB, ewB, isemB)

        @pl.when(w2 > 0)
        def _():
            pltpu.make_async_copy(rowsB, acc_sp.at[sidxB], ssB).wait()
        gb1, gb2 = gather_rows(idxB, rowsB, grB)
        fetch(e2 + 2, idxA, ewA, isemA)
        run_window(idxB, ewB, rowsB, sidxB, gb1, gb2)
        pltpu.async_copy(rowsB, acc_sp.at[sidxB], ssB, add=True)
        return carry
    lax.fori_loop(0, NW2H, body, 0)
    wait_fetch(idxA, ewA, isemA)
    pltpu.make_async_copy(rowsA, acc_sp.at[sidxA], ssA).wait()
    ga1, ga2 = gather_rows(idxA, rowsA, grA)
    run_window(idxA, ewA, rowsA, sidxA, ga1, ga2)
    pltpu.async_copy(rowsA, acc_sp.at[sidxA], ssA, add=True)
    pltpu.make_async_copy(rowsA, acc_sp.at[sidxA], ssA).wait()
    pltpu.make_async_copy(rowsB, acc_sp.at[sidxB], ssB).wait()
    plsc.subcore_barrier()

    @pl.when(cid == 0)
    def _():
        _row_chunks(sid, lambda off, sz: pltpu.sync_copy(
            acc_sp.at[pl.ds(off, sz)], outa_hbm.at[pl.ds(off, sz)]))

    @pl.when(cid == 1)
    def _():
        _row_chunks(sid, lambda off, sz: pltpu.sync_copy(
            acc_sp.at[pl.ds(off, sz)], outb_hbm.at[pl.ds(off, sz)]))


NWH = NW6 // 2


@functools.partial(
    pl.kernel,
    out_type=(jax.ShapeDtypeStruct((H, N, D), jnp.float32),
              jax.ShapeDtypeStruct((H * N,), jnp.float32)),
    mesh=_mesh,
    scratch_types=[
        pltpu.VMEM_SHARED((N, D), jnp.float32),
        pltpu.VMEM_SHARED((N,), jnp.float32),
        pltpu.VMEM_SHARED((N,), jnp.float32),
        pltpu.VMEM_SHARED((N,), jnp.float32),
        pltpu.VMEM((2, W), jnp.int32),
        pltpu.VMEM((2, W), jnp.int32),
        pltpu.VMEM((W,), jnp.int32),
        pltpu.VMEM((W,), jnp.int32),
        pltpu.VMEM((W, D), jnp.float32),
        pltpu.VMEM((W, D), jnp.float32),
        pltpu.VMEM((W,), jnp.float32),
        pltpu.VMEM((W,), jnp.float32),
        pltpu.VMEM((W,), jnp.float32),
        pltpu.VMEM((W,), jnp.float32),
        pltpu.VMEM((W,), jnp.float32),
        pltpu.VMEM((W,), jnp.float32),
        pltpu.VMEM((16,), jnp.float32),
        pltpu.VMEM((RCH + 16,), jnp.float32),
        pltpu.SemaphoreType.DMA,
        pltpu.SemaphoreType.DMA,
        pltpu.SemaphoreType.DMA,
        pltpu.SemaphoreType.DMA,
        pltpu.SemaphoreType.DMA,
        pltpu.SemaphoreType.DMA,
        pltpu.SemaphoreType.DMA,
        pltpu.SemaphoreType.DMA,
        pltpu.SemaphoreType.DMA,
        pltpu.SemaphoreType.DMA,
        pltpu.SemaphoreType.DMA,
        pltpu.SemaphoreType.DMA,
    ],
)
def _gat_kernel(h0_hbm, elt_hbm, ert_hbm, m_hbm, src_hbm, dst_hbm,
                zeros_hbm, zerosn_hbm, out_hbm, deno_hbm,
                acc_sp, el1_sp, er1_sp, den_sp,
                idxA, idxB, sidxA, sidxB, rowsA, rowsB,
                elbA, elbB, erbA, erbB, albA, albB, mv, bnc,
                isemA, isemB, grA, grB, geA, geB, gfA, gfB, ssA, ssB,
                sdA, sdB):
    cid = lax.axis_index("c")
    sid = lax.axis_index("s")

    def win_base(w):
        return pl.multiple_of(sid * ET6 + w * W, 8)

    def alpha_part(idx2, elb, erb, sidx, alb, mhead):
        def chunkf(k, c2):
            csl = pl.ds(k * 16, 16)
            z = elb[csl] + erb[csl]
            e = jnp.maximum(z, z * 0.2)
            alb[csl] = jnp.exp(e - mhead)
            sidx[csl] = idx2[1, csl]
            return c2
        lax.fori_loop(0, W // 16, chunkf, 0)

    def scale_part(rows, alb, k0, k1):
        def scalef(k, c2):
            a16 = alb[pl.ds(k * 16, 16)]
            for i in range(16):
                a = a16[i]
                row = k * 16 + i
                for j in range(D // 16):
                    sl = pl.ds(j * 16, 16)
                    rows[row, sl] = rows[row, sl] * a
            return c2
        lax.fori_loop(k0, k1, scalef, 0)

    for p in range(HPC):
        head = cid * HPC + p
        hoff = pl.multiple_of(head * N, 8)
        pltpu.sync_copy(m_hbm.at[pl.ds(pl.multiple_of(head * 16, 8), 16)],
                        mv)
        _row_chunks(sid, lambda off, sz: pltpu.sync_copy(
            zeros_hbm.at[pl.ds(off, sz)], acc_sp.at[pl.ds(off, sz)]))

        def zden(off, sz):
            pltpu.sync_copy(zerosn_hbm.at[pl.ds(off, sz)],
                            bnc.at[pl.ds(0, sz)])
            pltpu.sync_copy(bnc.at[pl.ds(0, sz)],
                            den_sp.at[pl.ds(off, sz)])
        _row_chunks(sid, zden)

        def stage(tbl_hbm, dst_sp):
            def cp(off, sz):
                pltpu.sync_copy(tbl_hbm.at[pl.ds(hoff + off, sz)],
                                bnc.at[pl.ds(0, sz)])
                pltpu.sync_copy(bnc.at[pl.ds(0, sz)],
                                dst_sp.at[pl.ds(off, sz)])
            _row_chunks(sid, cp)
        stage(elt_hbm, el1_sp)
        stage(ert_hbm, er1_sp)
        plsc.subcore_barrier()
        mhead = mv[:]

        def fetch_idx(wi, buf, sem):
            b = win_base(wi)
            pltpu.async_copy(src_hbm.at[pl.ds(b, W)], buf.at[0], sem)
            pltpu.async_copy(dst_hbm.at[pl.ds(b, W)], buf.at[1], sem)

        def wait_idx(buf, sem):
            pltpu.make_async_copy(
                src_hbm.at[pl.ds(0, W)], buf.at[0], sem).wait()
            pltpu.make_async_copy(
                dst_hbm.at[pl.ds(0, W)], buf.at[1], sem).wait()

        fetch_idx(0, idxA, isemA)

        def body(w2, carry):
            e2 = w2 * 2
            wait_idx(idxA, isemA)

            @pl.when(w2 > 0)
            def _():
                pltpu.make_async_copy(
                    rowsA, acc_sp.at[sidxA], ssA).wait()
                pltpu.make_async_copy(
                    albA, den_sp.at[sidxA], sdA).wait()
            ga1 = pltpu.async_copy(h0_hbm.at[idxA.at[0, pl.ds(0, 48)]],
                                   rowsA.at[pl.ds(0, 48)], grA)
            ga2 = pltpu.async_copy(h0_hbm.at[idxA.at[0, pl.ds(48, 32)]],
                                   rowsA.at[pl.ds(48, 32)], grA)
            gb = pltpu.async_copy(el1_sp.at[idxA.at[0]], elbA, geA)
            gc = pltpu.async_copy(er1_sp.at[idxA.at[1]], erbA, gfA)
            fetch_idx(e2 + 1, idxB, isemB)
            gb.wait()
            gc.wait()
            alpha_part(idxA, elbA, erbA, sidxA, albA, mhead)
            ga1.wait()
            scale_part(rowsA, albA, 0, 3)
            ga2.wait()
            scale_part(rowsA, albA, 3, W // 16)
            pltpu.async_copy(rowsA, acc_sp.at[sidxA], ssA, add=True)
            pltpu.async_copy(albA, den_sp.at[sidxA], sdA, add=True)
            wait_idx(idxB, isemB)

            @pl.when(w2 > 0)
            def _():
                pltpu.make_async_copy(
                    rowsB, acc_sp.at[sidxB], ssB).wait()
                pltpu.make_async_copy(
                    albB, den_sp.at[sidxB], sdB).wait()
            gd1 = pltpu.async_copy(h0_hbm.at[idxB.at[0, pl.ds(0, 48)]],
                                   rowsB.at[pl.ds(0, 48)], grB)
            gd2 = pltpu.async_copy(h0_hbm.at[idxB.at[0, pl.ds(48, 32)]],
                                   rowsB.at[pl.ds(48, 32)], grB)
            gb2 = pltpu.async_copy(el1_sp.at[idxB.at[0]], elbB, geB)
            gc2 = pltpu.async_copy(er1_sp.at[idxB.at[1]], erbB, gfB)

            @pl.when(w2 < NWH - 1)
            def _():
                fetch_idx(e2 + 2, idxA, isemA)
            gb2.wait()
            gc2.wait()
            alpha_part(idxB, elbB, erbB, sidxB, albB, mhead)
            gd1.wait()
            scale_part(rowsB, albB, 0, 3)
            gd2.wait()
            scale_part(rowsB, albB, 3, W // 16)
            pltpu.async_copy(rowsB, acc_sp.at[sidxB], ssB, add=True)
            pltpu.async_copy(albB, den_sp.at[sidxB], sdB, add=True)
            return carry
        lax.fori_loop(0, NWH, body, 0)
        pltpu.make_async_copy(rowsA, acc_sp.at[sidxA], ssA).wait()
        pltpu.make_async_copy(rowsB, acc_sp.at[sidxB], ssB).wait()
        pltpu.make_async_copy(albA, den_sp.at[sidxA], sdA).wait()
        pltpu.make_async_copy(albB, den_sp.at[sidxB], sdB).wait()
        plsc.subcore_barrier()
        _row_chunks(sid, lambda off, sz: pltpu.sync_copy(
            acc_sp.at[pl.ds(off, sz)], out_hbm.at[head, pl.ds(off, sz)]))

        def rden(off, sz):
            pltpu.sync_copy(den_sp.at[pl.ds(off, sz)],
                            bnc.at[pl.ds(0, sz)])
            pltpu.sync_copy(bnc.at[pl.ds(0, sz)],
                            deno_hbm.at[pl.ds(hoff + off, sz)])
        _row_chunks(sid, rden)
        plsc.subcore_barrier()


def _mm_body(x_ref, w_ref, o_ref):
    o_ref[...] = jnp.dot(x_ref[...], w_ref[...],
                         preferred_element_type=jnp.float32)


def _t1(x, W1):
    return pl.pallas_call(
        _mm_body,
        grid=(10,),
        in_specs=[pl.BlockSpec((N // 10, D), lambda i: (i, 0)),
                  pl.BlockSpec((D, D), lambda i: (0, 0))],
        out_specs=pl.BlockSpec((N // 10, D), lambda i: (i, 0)),
        out_shape=jax.ShapeDtypeStruct((N, D), jnp.float32),
    )(x, W1)


def _t3_body(aggA, aggB, b1, W2, al, ar, h0_o, elt_o, ert_o, m_o):
    h0 = jnp.maximum(aggA[...] + aggB[...] + b1[...], 0.0)
    h0_o[...] = h0
    w2 = W2[...]
    alv = al[...]
    arv = ar[...]
    cols_l = []
    cols_r = []
    for h in range(H):
        w2h = w2[:, h * O:(h + 1) * O]
        cols_l.append(lax.dot_general(w2h, alv[h:h + 1, :],
                                      (((1,), (1,)), ((), ()))))
        cols_r.append(lax.dot_general(w2h, arv[h:h + 1, :],
                                      (((1,), (1,)), ((), ()))))
    Wl = jnp.concatenate(cols_l, axis=1)
    Wr = jnp.concatenate(cols_r, axis=1)
    elt = lax.dot_general(Wl, h0, (((0,), (1,)), ((), ())))
    ert = lax.dot_general(Wr, h0, (((0,), (1,)), ((), ())))
    elt_o[...] = elt
    ert_o[...] = ert
    m = (jnp.max(elt, axis=1, keepdims=True)
         + jnp.max(ert, axis=1, keepdims=True))
    m_o[...] = jnp.broadcast_to(m, (H, 16))


def _t3(aggA, aggB, b1, W2, al, ar):
    return pl.pallas_call(
        _t3_body,
        out_shape=(jax.ShapeDtypeStruct((N, D), jnp.float32),
                   jax.ShapeDtypeStruct((H, N), jnp.float32),
                   jax.ShapeDtypeStruct((H, N), jnp.float32),
                   jax.ShapeDtypeStruct((H, 16), jnp.float32)),
    )(aggA, aggB, b1, W2, al, ar)


def _t7_body(acc_ref, ur_ref, W2_ref, b2_ref, o_ref):
    w2 = W2_ref[...]
    b2 = b2_ref[...]
    acc = acc_ref[...]
    ur = 1.0 / jnp.clip(ur_ref[...], 1e-9, None)
    BN = acc.shape[1]
    s = jnp.zeros((BN, D), jnp.float32)
    for h in range(H):
        y = jnp.dot(acc[h], w2[:, h * O:(h + 1) * O],
                    preferred_element_type=jnp.float32)
        y = y * ur[:, h:h + 1] + b2[:, h * O:(h + 1) * O]
        s = s + jnp.maximum(y, 0.0)
    o_ref[...] = s * (1.0 / H)


def _t7(acc8, urec, W2, b2):
    BN = N // 10
    return pl.pallas_call(
        _t7_body,
        grid=(10,),
        in_specs=[pl.BlockSpec((H, BN, D), lambda i: (0, i, 0)),
                  pl.BlockSpec((BN, H), lambda i: (i, 0)),
                  pl.BlockSpec((D, H * O), lambda i: (0, 0)),
                  pl.BlockSpec((1, H * O), lambda i: (0, 0))],
        out_specs=pl.BlockSpec((BN, D), lambda i: (i, 0)),
        out_shape=jax.ShapeDtypeStruct((N, D), jnp.float32),
    )(acc8, urec, W2, b2)


def kernel(x, edge_index, edge_weight, W1, b1, W2, attn_l, attn_r, b2):
    src = edge_index[0]
    dst = edge_index[1]
    zeros_nd = jnp.zeros((N, D), jnp.float32)
    zeros_n = jnp.zeros((N,), jnp.float32)

    h = _t1(x, W1)
    aggA, aggB = _agg0_kernel(h, src, dst, edge_weight, zeros_nd)
    h0, elt, ert, mb = _t3(aggA, aggB, b1.reshape(1, D), W2,
                           attn_l, attn_r)
    m16 = mb.reshape(H * 16)
    elt_f = elt.reshape(H * N)
    ert_f = ert.reshape(H * N)
    acc8, den_f = _gat_kernel(h0, elt_f, ert_f, m16, src, dst,
                              zeros_nd, zeros_n)
    out = _t7(acc8, den_f.reshape(H, N).T, W2, b2.reshape(1, H * O))
    return out
